# Initial kernel scaffold; baseline (speedup 1.0000x reference)
import jax
import jax.numpy as jnp
from jax.experimental import pallas as pl


def kernel(edge_index, attr_mtx, x, params):
    raise NotImplementedError("write your pallas kernel here")


# speedup vs baseline: 1.0003x; 1.0003x over previous
import functools

import jax
import jax.numpy as jnp
from jax.experimental import pallas as pl
from jax.experimental.pallas import tpu as pltpu

N = 50000
E = 800000
H = 2
B = 16384


def _pair_mlp_body(xc, w1, b1, g1, be1, w2, b2, g2, be2, w3, b3, o):
    s = 1.0 / jnp.sqrt(1.0 + 1e-5)
    h = jnp.maximum(xc[...] @ w1[...] + b1[...], 0.0)
    h = h * s * g1[...] + be1[...]
    h = jnp.maximum(h @ w2[...] + b2[...], 0.0)
    h = h * s * g2[...] + be2[...]
    o[...] = jax.nn.sigmoid(h @ w3[...] + b3[...])


def _pair_mlp(xc, p):
    blk = 2048
    grid = (B // blk,)
    return pl.pallas_call(
        _pair_mlp_body,
        grid=grid,
        in_specs=[
            pl.BlockSpec((blk, 256), lambda i: (i, 0)),
            pl.BlockSpec((256, 512), lambda i: (0, 0)),
            pl.BlockSpec((512,), lambda i: (0,)),
            pl.BlockSpec((512,), lambda i: (0,)),
            pl.BlockSpec((512,), lambda i: (0,)),
            pl.BlockSpec((512, 256), lambda i: (0, 0)),
            pl.BlockSpec((256,), lambda i: (0,)),
            pl.BlockSpec((256,), lambda i: (0,)),
            pl.BlockSpec((256,), lambda i: (0,)),
            pl.BlockSpec((256, 1), lambda i: (0, 0)),
            pl.BlockSpec((1,), lambda i: (0,)),
        ],
        out_specs=pl.BlockSpec((blk, 1), lambda i: (i, 0)),
        out_shape=jax.ShapeDtypeStruct((B, 1), jnp.float32),
    )(xc, p['Wd1'], p['bd1'], p['g1'], p['be1'],
      p['Wd2'], p['bd2'], p['g2'], p['be2'], p['Wd3'], p['bd3'])


def _gat_layer(x, edge_index, W, a_src, a_dst, bias, n, h, c):
    loop = jnp.arange(n, dtype=edge_index.dtype)
    src = jnp.concatenate([edge_index[0], loop])
    dst = jnp.concatenate([edge_index[1], loop])
    feat = (x @ W).reshape(n, h, c)
    alpha_src = (feat * a_src[None, :, :]).sum(-1)
    alpha_dst = (feat * a_dst[None, :, :]).sum(-1)
    alpha = alpha_src[src] + alpha_dst[dst]
    alpha = jnp.where(alpha > 0, alpha, 0.2 * alpha)
    amax = jax.ops.segment_max(alpha, dst, num_segments=n)
    ex = jnp.exp(alpha - amax[dst])
    den = jax.ops.segment_sum(ex, dst, num_segments=n)
    att = ex / (den[dst] + 1e-16)
    msg = feat[src] * att[:, :, None]
    out = jax.ops.segment_sum(msg, dst, num_segments=n)
    return out.mean(axis=1) + bias


def kernel(edge_index, attr_mtx, x_pairs, p):
    h1 = _gat_layer(p['X'], edge_index, p['W1'], p['as1'], p['ad1'], p['b1'], N, H, 64)
    gcn_out = _gat_layer(h1, edge_index, p['W2'], p['as2'], p['ad2'], p['b2'], N, H, 64)
    topo = (gcn_out @ p['Wt1'] + p['bt1']) @ p['Wt2'] + p['bt2']
    t2a = (topo @ p['Wta1'] + p['bta1']) @ p['Wta2'] + p['bta2']
    attr = (attr_mtx @ p['Wa1'] + p['ba1']) @ p['Wa2'] + p['ba2']
    a2t = (attr @ p['Wat1'] + p['bat1']) @ p['Wat2'] + p['bat2']
    emb = jnp.concatenate([topo, attr], axis=1)
    Xc = jnp.concatenate([emb[x_pairs[:, 0]], emb[x_pairs[:, 1]]], axis=1)
    out = _pair_mlp(Xc, p)
    return (out, gcn_out, t2a, a2t)


# speedup vs baseline: 23.9085x; 23.9013x over previous
import functools

import jax
import jax.numpy as jnp
from jax import lax
from jax.experimental import pallas as pl
from jax.experimental.pallas import tpu as pltpu
from jax.experimental.pallas import tpu_sc as plsc

N = 50000
E = 800000
B = 16384
EP = 819200
NW = 32
EW = EP // NW
BK = 1024
NBLK = EW // BK
RPT = EW // 128
RPB = BK // 128

_f32 = jnp.float32
_i32 = jnp.int32


def _sc_gat_body(src_h, dst_h, asrc_h, adst_h, f0_h, f1_h, f2_h, f3_h,
                 f4_h, f5_h, f6_h, f7_h, z2_h, z16_h,
                 den_o, acc_o, ex_o,
                 den_sp, acc_sp, src_i, dst_i, vsrc_i, vdst_i, sg, dg,
                 exblk, rows):
    c = lax.axis_index("c")
    s = lax.axis_index("s")
    wid = s * 2 + c
    lane = jnp.arange(16, dtype=_i32)
    half = lane >> 1
    par = lane & 1

    @pl.when(s == 0)
    def _():
        pltpu.sync_copy(z2_h, den_sp)
    plsc.subcore_barrier()

    def blk_a(blk, carry):
        ebase = wid * EW + blk * BK
        pltpu.sync_copy(src_h.at[pl.ds(ebase, BK)], src_i)
        pltpu.sync_copy(dst_h.at[pl.ds(ebase, BK)], dst_i)

        def mkidx(i, carry2):
            eloc = i * 8 + half
            vsrc_i[pl.ds(i * 16, 16)] = (
                plsc.load_gather(src_i, [eloc]) * 2 + par)
            vdst_i[pl.ds(i * 16, 16)] = (
                plsc.load_gather(dst_i, [eloc]) * 2 + par)
            return carry2
        lax.fori_loop(0, 2 * BK // 16, mkidx, 0)
        pltpu.sync_copy(asrc_h.at[vsrc_i], sg)
        pltpu.sync_copy(adst_h.at[vdst_i], dg)

        def cmp16(i, carry2):
            sl = pl.ds(i * 16, 16)
            a = sg[sl] + dg[sl]
            a = jnp.where(a > 0, a, 0.2 * a)
            ev = jnp.exp(a)
            ev = jnp.where(ebase + i * 8 + half < E, ev, 0.0)
            exblk[sl] = ev
            return carry2
        lax.fori_loop(0, 2 * BK // 16, cmp16, 0)
        pltpu.sync_copy(exblk, den_sp.at[vdst_i], add=True)
        pltpu.sync_copy(exblk, ex_o.at[pl.ds(2 * ebase, 2 * BK)])
        return carry
    lax.fori_loop(0, NBLK, blk_a, 0)

    plsc.subcore_barrier()

    @pl.when(s == 0)
    def _():
        pltpu.sync_copy(den_sp, den_o.at[c])

    for chunk in range(8):
        h = chunk // 4
        fc_h = (f0_h, f1_h, f2_h, f3_h, f4_h, f5_h, f6_h, f7_h)[chunk]

        @pl.when(s == 0)
        def _():
            pltpu.sync_copy(z16_h, acc_sp)
        plsc.subcore_barrier()

        def blk_b(blk, carry, fc_h=fc_h, h=h):
            ebase = wid * EW + blk * BK
            pltpu.sync_copy(src_h.at[pl.ds(ebase, BK)], src_i)
            pltpu.sync_copy(dst_h.at[pl.ds(ebase, BK)], dst_i)
            pltpu.sync_copy(fc_h.at[src_i], rows)
            pltpu.sync_copy(ex_o.at[pl.ds(2 * ebase, 2 * BK)], exblk)

            def edge(e, carry2):
                exv = plsc.load_gather(
                    exblk, [jnp.full((16,), 2 * e + h, dtype=_i32)])
                rows[e, :] = rows[e, :] * exv
                return carry2
            lax.fori_loop(0, BK, edge, 0)
            pltpu.sync_copy(rows, acc_sp.at[dst_i], add=True)
            return carry
        lax.fori_loop(0, NBLK, blk_b, 0)

        plsc.subcore_barrier()

        @pl.when(s == 0)
        def _(chunk=chunk):
            pltpu.sync_copy(acc_sp, acc_o.at[c * 8 + chunk])


def _sc_gat(srcp, dstp, asrc, adst, fcs, z2, z16):
    fn = pl.kernel(
        _sc_gat_body,
        out_type=(jax.ShapeDtypeStruct((2, 2 * N), _f32),
                  jax.ShapeDtypeStruct((16, N, 16), _f32),
                  jax.ShapeDtypeStruct((2 * EP,), _f32)),
        mesh=plsc.VectorSubcoreMesh(core_axis_name="c", subcore_axis_name="s"),
        scratch_types=[
            pltpu.VMEM_SHARED((2 * N,), _f32),
            pltpu.VMEM_SHARED((N, 16), _f32),
            pltpu.VMEM((BK,), _i32),
            pltpu.VMEM((BK,), _i32),
            pltpu.VMEM((2 * BK,), _i32),
            pltpu.VMEM((2 * BK,), _i32),
            pltpu.VMEM((2 * BK,), _f32),
            pltpu.VMEM((2 * BK,), _f32),
            pltpu.VMEM((2 * BK,), _f32),
            pltpu.VMEM((BK, 16), _f32),
        ],
        compiler_params=pltpu.CompilerParams(needs_layout_passes=False,
                                             use_tc_tiling_on_sc=False),
    )
    den, acc, _ex = fn(srcp, dstp, asrc, adst, *fcs, z2, z16)
    return den, acc


def _sc_pair_body(topo_h, attr_h, xl_h, xr_h, tl_o, al_o, tr_o, ar_o,
                  pidx, prow):
    c = lax.axis_index("c")
    s = lax.axis_index("s")
    wid = s * 2 + c
    base = wid * (B // NW)
    pltpu.sync_copy(xl_h.at[pl.ds(base, B // NW)], pidx)
    pltpu.sync_copy(topo_h.at[pidx], prow)
    pltpu.sync_copy(prow, tl_o.at[pl.ds(base, B // NW)])
    pltpu.sync_copy(attr_h.at[pidx], prow)
    pltpu.sync_copy(prow, al_o.at[pl.ds(base, B // NW)])
    pltpu.sync_copy(xr_h.at[pl.ds(base, B // NW)], pidx)
    pltpu.sync_copy(topo_h.at[pidx], prow)
    pltpu.sync_copy(prow, tr_o.at[pl.ds(base, B // NW)])
    pltpu.sync_copy(attr_h.at[pidx], prow)
    pltpu.sync_copy(prow, ar_o.at[pl.ds(base, B // NW)])


def _sc_pair(topo, attr, xl, xr):
    fn = pl.kernel(
        _sc_pair_body,
        out_type=tuple(jax.ShapeDtypeStruct((B, 64), _f32) for _ in range(4)),
        mesh=plsc.VectorSubcoreMesh(core_axis_name="c", subcore_axis_name="s"),
        scratch_types=[
            pltpu.VMEM((B // NW,), _i32),
            pltpu.VMEM((B // NW, 64), _f32),
        ],
        compiler_params=pltpu.CompilerParams(needs_layout_passes=False,
                                             use_tc_tiling_on_sc=False),
    )
    return fn(topo, attr, xl, xr)


_NB = 2000


def _feat_body(x_r, w_r, as_r, ad_r, *rest):
    f_rs = rest[0:8]
    s_r, d_r = rest[8], rest[9]
    f = jnp.dot(x_r[...], w_r[...], preferred_element_type=_f32)
    for g in range(8):
        f_rs[g][...] = f[:, 16 * g:16 * (g + 1)]
    s_r[...] = jnp.dot(f, as_r[...], preferred_element_type=_f32)
    d_r[...] = jnp.dot(f, ad_r[...], preferred_element_type=_f32)


def _tc_feat(x, w, a_s, a_d):
    fin = x.shape[1]
    outs = pl.pallas_call(
        _feat_body,
        grid=(N // _NB,),
        in_specs=[
            pl.BlockSpec((_NB, fin), lambda i: (i, 0)),
            pl.BlockSpec((fin, 128), lambda i: (0, 0)),
            pl.BlockSpec((128, 2), lambda i: (0, 0)),
            pl.BlockSpec((128, 2), lambda i: (0, 0)),
        ],
        out_specs=[pl.BlockSpec((_NB, 16), lambda i: (i, 0))] * 8
        + [pl.BlockSpec((_NB, 2), lambda i: (i, 0))] * 2,
        out_shape=[jax.ShapeDtypeStruct((N, 16), _f32)] * 8
        + [jax.ShapeDtypeStruct((N, 2), _f32)] * 2,
    )(x, w, a_s, a_d)
    return outs[0:8], outs[8], outs[9]


def _epi_body(acc_r, den_r, *rest):
    f_rs = rest[0:8]
    s_r, d_r, b_r, o_r = rest[8], rest[9], rest[10], rest[11]
    a = s_r[...] + d_r[...]
    exs = jnp.exp(jnp.where(a > 0, a, 0.2 * a))
    den = den_r[0] + den_r[1] + exs + 1e-16
    for q in range(4):
        n0 = acc_r[q] + acc_r[8 + q] + f_rs[q][...] * exs[:, 0:1]
        n1 = acc_r[4 + q] + acc_r[12 + q] + f_rs[4 + q][...] * exs[:, 1:2]
        o_r[:, q * 16:(q + 1) * 16] = (
            0.5 * (n0 / den[:, 0:1] + n1 / den[:, 1:2])
            + b_r[q * 16:(q + 1) * 16])


def _tc_epi(acc, den, fcs, asrc, adst, bias):
    nb = 1000
    return pl.pallas_call(
        _epi_body,
        grid=(N // nb,),
        in_specs=[
            pl.BlockSpec((16, nb, 16), lambda i: (0, i, 0)),
            pl.BlockSpec((2, nb, 2), lambda i: (0, i, 0)),
        ]
        + [pl.BlockSpec((nb, 16), lambda i: (i, 0))] * 8
        + [pl.BlockSpec((nb, 2), lambda i: (i, 0))] * 2
        + [pl.BlockSpec((64,), lambda i: (0,))],
        out_specs=pl.BlockSpec((nb, 64), lambda i: (i, 0)),
        out_shape=jax.ShapeDtypeStruct((N, 64), _f32),
    )(acc, den, *fcs, asrc, adst, bias)


def _mlp2_body(x_r, w1_r, b1_r, w2_r, b2_r, w3_r, b3_r, w4_r, b4_r,
               y2_r, y4_r):
    t = jnp.dot(x_r[...], w1_r[...], preferred_element_type=_f32) + b1_r[...]
    y2 = jnp.dot(t, w2_r[...], preferred_element_type=_f32) + b2_r[...]
    y2_r[...] = y2
    u = jnp.dot(y2, w3_r[...], preferred_element_type=_f32) + b3_r[...]
    y4_r[...] = jnp.dot(u, w4_r[...], preferred_element_type=_f32) + b4_r[...]


def _tc_mlp2(x, w1, b1, w2, b2, w3, b3, w4, b4, dmid, dout):
    fin = x.shape[1]
    return pl.pallas_call(
        _mlp2_body,
        grid=(N // _NB,),
        in_specs=[
            pl.BlockSpec((_NB, fin), lambda i: (i, 0)),
            pl.BlockSpec((fin, 100), lambda i: (0, 0)),
            pl.BlockSpec((100,), lambda i: (0,)),
            pl.BlockSpec((100, dmid), lambda i: (0, 0)),
            pl.BlockSpec((dmid,), lambda i: (0,)),
            pl.BlockSpec((dmid, 100), lambda i: (0, 0)),
            pl.BlockSpec((100,), lambda i: (0,)),
            pl.BlockSpec((100, dout), lambda i: (0, 0)),
            pl.BlockSpec((dout,), lambda i: (0,)),
        ],
        out_specs=[
            pl.BlockSpec((_NB, dmid), lambda i: (i, 0)),
            pl.BlockSpec((_NB, dout), lambda i: (i, 0)),
        ],
        out_shape=[
            jax.ShapeDtypeStruct((N, dmid), _f32),
            jax.ShapeDtypeStruct((N, dout), _f32),
        ],
    )(x, w1, b1, w2, b2, w3, b3, w4, b4)


_BB = 2048


def _cls_body(tl_r, al_r, tr_r, ar_r, w1a_r, w1b_r, w1c_r, w1d_r, b1_r,
              g1_r, be1_r, w2_r, b2_r, g2_r, be2_r, w3_r, b3_r, o_r):
    s = 1.0 / jnp.sqrt(1.0 + 1e-5)
    h = (jnp.dot(tl_r[...], w1a_r[...], preferred_element_type=_f32)
         + jnp.dot(al_r[...], w1b_r[...], preferred_element_type=_f32)
         + jnp.dot(tr_r[...], w1c_r[...], preferred_element_type=_f32)
         + jnp.dot(ar_r[...], w1d_r[...], preferred_element_type=_f32)
         + b1_r[...])
    h = jnp.maximum(h, 0.0) * s * g1_r[...] + be1_r[...]
    h = jnp.dot(h, w2_r[...], preferred_element_type=_f32) + b2_r[...]
    h = jnp.maximum(h, 0.0) * s * g2_r[...] + be2_r[...]
    o_r[...] = jax.nn.sigmoid(
        jnp.dot(h, w3_r[...], preferred_element_type=_f32) + b3_r[...])


def _tc_cls(tl, al, tr, ar, p):
    w1 = p['Wd1']
    return pl.pallas_call(
        _cls_body,
        grid=(B // _BB,),
        in_specs=[pl.BlockSpec((_BB, 64), lambda i: (i, 0))] * 4
        + [pl.BlockSpec((64, 512), lambda i: (0, 0))] * 4
        + [pl.BlockSpec((512,), lambda i: (0,))] * 3
        + [pl.BlockSpec((512, 256), lambda i: (0, 0))]
        + [pl.BlockSpec((256,), lambda i: (0,))] * 3
        + [pl.BlockSpec((256, 1), lambda i: (0, 0)),
           pl.BlockSpec((1,), lambda i: (0,))],
        out_specs=pl.BlockSpec((_BB, 1), lambda i: (i, 0)),
        out_shape=jax.ShapeDtypeStruct((B, 1), _f32),
    )(tl, al, tr, ar, w1[0:64], w1[64:128], w1[128:192], w1[192:256],
      p['bd1'], p['g1'], p['be1'], p['Wd2'], p['bd2'], p['g2'], p['be2'],
      p['Wd3'], p['bd3'])


def _alpha_mats(a_src, a_dst):
    a_s = jnp.zeros((128, 2), _f32).at[0:64, 0].set(a_src[0]).at[64:128, 1].set(a_src[1])
    a_d = jnp.zeros((128, 2), _f32).at[0:64, 0].set(a_dst[0]).at[64:128, 1].set(a_dst[1])
    return a_s, a_d


def _gat_layer(x, srcp, dstp, w, a_src, a_dst, bias, z2, z16):
    a_s, a_d = _alpha_mats(a_src, a_dst)
    fcs, asrc, adst = _tc_feat(x, w, a_s, a_d)
    den, acc = _sc_gat(srcp, dstp, asrc.reshape(-1), adst.reshape(-1),
                       fcs, z2, z16)
    return _tc_epi(acc, den.reshape(2, N, 2), fcs, asrc, adst, bias)


def kernel(edge_index, attr_mtx, x_pairs, p):
    pad = jnp.zeros((EP - E,), _i32)
    srcp = jnp.concatenate([edge_index[0], pad])
    dstp = jnp.concatenate([edge_index[1], pad])
    xl = x_pairs[:, 0]
    xr = x_pairs[:, 1]
    z2 = jnp.zeros((2 * N,), _f32)
    z16 = jnp.zeros((N, 16), _f32)

    h1 = _gat_layer(p['X'], srcp, dstp, p['W1'], p['as1'], p['ad1'],
                    p['b1'], z2, z16)
    gcn_out = _gat_layer(h1, srcp, dstp, p['W2'], p['as2'], p['ad2'],
                         p['b2'], z2, z16)

    attr_emb, a2t = _tc_mlp2(attr_mtx, p['Wa1'], p['ba1'], p['Wa2'],
                             p['ba2'], p['Wat1'], p['bat1'], p['Wat2'],
                             p['bat2'], 64, 64)
    topo, t2a = _tc_mlp2(gcn_out, p['Wt1'], p['bt1'], p['Wt2'], p['bt2'],
                         p['Wta1'], p['bta1'], p['Wta2'], p['bta2'], 64, 128)

    tl, al, tr, ar = _sc_pair(topo, attr_emb, xl, xr)
    out = _tc_cls(tl, al, tr, ar, p)
    return (out, gcn_out, t2a, a2t)


# speedup vs baseline: 25.2122x; 1.0545x over previous
import functools

import jax
import jax.numpy as jnp
from jax import lax
from jax.experimental import pallas as pl
from jax.experimental.pallas import tpu as pltpu
from jax.experimental.pallas import tpu_sc as plsc

N = 50000
E = 800000
B = 16384
EP = 819200
NW = 32
EW = EP // NW
BK = 1600
NBLK = EW // BK
RPT = EW // 128
RPB = BK // 128

_f32 = jnp.float32
_i32 = jnp.int32


def _sc_gat_body(src_h, dst_h, asrc_h, adst_h, f0_h, f1_h, f2_h, f3_h,
                 f4_h, f5_h, f6_h, f7_h, z2_h, z16_h,
                 den_o, acc_o, ex_o,
                 den_sp, acc_sp, src_i, dst_i, vsrc_i, vdst_i, sg, dg,
                 exblk, rows):
    c = lax.axis_index("c")
    s = lax.axis_index("s")
    wid = s * 2 + c
    lane = jnp.arange(16, dtype=_i32)
    half = lane >> 1
    par = lane & 1

    @pl.when(s == 0)
    def _():
        pltpu.sync_copy(z2_h, den_sp)
    plsc.subcore_barrier()

    def blk_a(blk, carry):
        ebase = wid * EW + blk * BK
        pltpu.sync_copy(src_h.at[pl.ds(ebase, BK)], src_i)
        pltpu.sync_copy(dst_h.at[pl.ds(ebase, BK)], dst_i)

        def mkidx(i, carry2):
            eloc = i * 8 + half
            vsrc_i[pl.ds(i * 16, 16)] = (
                plsc.load_gather(src_i, [eloc]) * 2 + par)
            vdst_i[pl.ds(i * 16, 16)] = (
                plsc.load_gather(dst_i, [eloc]) * 2 + par)
            return carry2
        lax.fori_loop(0, 2 * BK // 16, mkidx, 0)
        pltpu.sync_copy(asrc_h.at[vsrc_i], sg)
        pltpu.sync_copy(adst_h.at[vdst_i], dg)

        def cmp16(i, carry2):
            sl = pl.ds(i * 16, 16)
            a = sg[sl] + dg[sl]
            a = jnp.where(a > 0, a, 0.2 * a)
            ev = jnp.exp(a)
            ev = jnp.where(ebase + i * 8 + half < E, ev, 0.0)
            exblk[sl] = ev
            return carry2
        lax.fori_loop(0, 2 * BK // 16, cmp16, 0)
        pltpu.sync_copy(exblk, den_sp.at[vdst_i], add=True)
        pltpu.sync_copy(exblk, ex_o.at[pl.ds(2 * ebase, 2 * BK)])
        return carry
    lax.fori_loop(0, NBLK, blk_a, 0)

    plsc.subcore_barrier()

    @pl.when(s == 0)
    def _():
        pltpu.sync_copy(den_sp, den_o.at[c])

    for chunk in range(8):
        h = chunk // 4
        fc_h = (f0_h, f1_h, f2_h, f3_h, f4_h, f5_h, f6_h, f7_h)[chunk]

        @pl.when(s == 0)
        def _():
            pltpu.sync_copy(z16_h, acc_sp)
        plsc.subcore_barrier()

        def blk_b(blk, carry, fc_h=fc_h, h=h):
            ebase = wid * EW + blk * BK
            pltpu.sync_copy(src_h.at[pl.ds(ebase, BK)], src_i)
            pltpu.sync_copy(dst_h.at[pl.ds(ebase, BK)], dst_i)
            pltpu.sync_copy(fc_h.at[src_i], rows)
            pltpu.sync_copy(ex_o.at[pl.ds(2 * ebase, 2 * BK)], exblk)

            def edge8(j, carry2):
                e0 = j * 8
                for u in range(8):
                    e = e0 + u
                    exv = plsc.load_gather(
                        exblk, [jnp.full((16,), 2 * e + h, dtype=_i32)])
                    rows[e, :] = rows[e, :] * exv
                return carry2
            lax.fori_loop(0, BK // 8, edge8, 0)
            pltpu.sync_copy(rows, acc_sp.at[dst_i], add=True)
            return carry
        lax.fori_loop(0, NBLK, blk_b, 0)

        plsc.subcore_barrier()

        @pl.when(s == 0)
        def _(chunk=chunk):
            pltpu.sync_copy(acc_sp, acc_o.at[c * 8 + chunk])


def _sc_gat(srcp, dstp, asrc, adst, fcs, z2, z16):
    fn = pl.kernel(
        _sc_gat_body,
        out_type=(jax.ShapeDtypeStruct((2, 2 * N), _f32),
                  jax.ShapeDtypeStruct((16, N, 16), _f32),
                  jax.ShapeDtypeStruct((2 * EP,), _f32)),
        mesh=plsc.VectorSubcoreMesh(core_axis_name="c", subcore_axis_name="s"),
        scratch_types=[
            pltpu.VMEM_SHARED((2 * N,), _f32),
            pltpu.VMEM_SHARED((N, 16), _f32),
            pltpu.VMEM((BK,), _i32),
            pltpu.VMEM((BK,), _i32),
            pltpu.VMEM((2 * BK,), _i32),
            pltpu.VMEM((2 * BK,), _i32),
            pltpu.VMEM((2 * BK,), _f32),
            pltpu.VMEM((2 * BK,), _f32),
            pltpu.VMEM((2 * BK,), _f32),
            pltpu.VMEM((BK, 16), _f32),
        ],
        compiler_params=pltpu.CompilerParams(needs_layout_passes=False,
                                             use_tc_tiling_on_sc=False),
    )
    den, acc, _ex = fn(srcp, dstp, asrc, adst, *fcs, z2, z16)
    return den, acc


def _sc_pair_body(topo_h, attr_h, xl_h, xr_h, tl_o, al_o, tr_o, ar_o,
                  pidx, prow):
    c = lax.axis_index("c")
    s = lax.axis_index("s")
    wid = s * 2 + c
    base = wid * (B // NW)
    pltpu.sync_copy(xl_h.at[pl.ds(base, B // NW)], pidx)
    pltpu.sync_copy(topo_h.at[pidx], prow)
    pltpu.sync_copy(prow, tl_o.at[pl.ds(base, B // NW)])
    pltpu.sync_copy(attr_h.at[pidx], prow)
    pltpu.sync_copy(prow, al_o.at[pl.ds(base, B // NW)])
    pltpu.sync_copy(xr_h.at[pl.ds(base, B // NW)], pidx)
    pltpu.sync_copy(topo_h.at[pidx], prow)
    pltpu.sync_copy(prow, tr_o.at[pl.ds(base, B // NW)])
    pltpu.sync_copy(attr_h.at[pidx], prow)
    pltpu.sync_copy(prow, ar_o.at[pl.ds(base, B // NW)])


def _sc_pair(topo, attr, xl, xr):
    fn = pl.kernel(
        _sc_pair_body,
        out_type=tuple(jax.ShapeDtypeStruct((B, 64), _f32) for _ in range(4)),
        mesh=plsc.VectorSubcoreMesh(core_axis_name="c", subcore_axis_name="s"),
        scratch_types=[
            pltpu.VMEM((B // NW,), _i32),
            pltpu.VMEM((B // NW, 64), _f32),
        ],
        compiler_params=pltpu.CompilerParams(needs_layout_passes=False,
                                             use_tc_tiling_on_sc=False),
    )
    return fn(topo, attr, xl, xr)


_NB = 2000


def _feat_body(x_r, w_r, as_r, ad_r, *rest):
    f_rs = rest[0:8]
    s_r, d_r = rest[8], rest[9]
    f = jnp.dot(x_r[...], w_r[...], preferred_element_type=_f32)
    for g in range(8):
        f_rs[g][...] = f[:, 16 * g:16 * (g + 1)]
    s_r[...] = jnp.dot(f, as_r[...], preferred_element_type=_f32)
    d_r[...] = jnp.dot(f, ad_r[...], preferred_element_type=_f32)


def _tc_feat(x, w, a_s, a_d):
    fin = x.shape[1]
    outs = pl.pallas_call(
        _feat_body,
        grid=(N // _NB,),
        in_specs=[
            pl.BlockSpec((_NB, fin), lambda i: (i, 0)),
            pl.BlockSpec((fin, 128), lambda i: (0, 0)),
            pl.BlockSpec((128, 2), lambda i: (0, 0)),
            pl.BlockSpec((128, 2), lambda i: (0, 0)),
        ],
        out_specs=[pl.BlockSpec((_NB, 16), lambda i: (i, 0))] * 8
        + [pl.BlockSpec((_NB, 2), lambda i: (i, 0))] * 2,
        out_shape=[jax.ShapeDtypeStruct((N, 16), _f32)] * 8
        + [jax.ShapeDtypeStruct((N, 2), _f32)] * 2,
    )(x, w, a_s, a_d)
    return outs[0:8], outs[8], outs[9]


def _epi_body(acc_r, den_r, *rest):
    f_rs = rest[0:8]
    s_r, d_r, b_r, o_r = rest[8], rest[9], rest[10], rest[11]
    a = s_r[...] + d_r[...]
    exs = jnp.exp(jnp.where(a > 0, a, 0.2 * a))
    den = den_r[0] + den_r[1] + exs + 1e-16
    for q in range(4):
        n0 = acc_r[q] + acc_r[8 + q] + f_rs[q][...] * exs[:, 0:1]
        n1 = acc_r[4 + q] + acc_r[12 + q] + f_rs[4 + q][...] * exs[:, 1:2]
        o_r[:, q * 16:(q + 1) * 16] = (
            0.5 * (n0 / den[:, 0:1] + n1 / den[:, 1:2])
            + b_r[q * 16:(q + 1) * 16])


def _tc_epi(acc, den, fcs, asrc, adst, bias):
    nb = 1000
    return pl.pallas_call(
        _epi_body,
        grid=(N // nb,),
        in_specs=[
            pl.BlockSpec((16, nb, 16), lambda i: (0, i, 0)),
            pl.BlockSpec((2, nb, 2), lambda i: (0, i, 0)),
        ]
        + [pl.BlockSpec((nb, 16), lambda i: (i, 0))] * 8
        + [pl.BlockSpec((nb, 2), lambda i: (i, 0))] * 2
        + [pl.BlockSpec((64,), lambda i: (0,))],
        out_specs=pl.BlockSpec((nb, 64), lambda i: (i, 0)),
        out_shape=jax.ShapeDtypeStruct((N, 64), _f32),
    )(acc, den, *fcs, asrc, adst, bias)


def _mlp2_body(x_r, w1_r, b1_r, w2_r, b2_r, w3_r, b3_r, w4_r, b4_r,
               y2_r, y4_r):
    t = jnp.dot(x_r[...], w1_r[...], preferred_element_type=_f32) + b1_r[...]
    y2 = jnp.dot(t, w2_r[...], preferred_element_type=_f32) + b2_r[...]
    y2_r[...] = y2
    u = jnp.dot(y2, w3_r[...], preferred_element_type=_f32) + b3_r[...]
    y4_r[...] = jnp.dot(u, w4_r[...], preferred_element_type=_f32) + b4_r[...]


def _tc_mlp2(x, w1, b1, w2, b2, w3, b3, w4, b4, dmid, dout):
    fin = x.shape[1]
    return pl.pallas_call(
        _mlp2_body,
        grid=(N // _NB,),
        in_specs=[
            pl.BlockSpec((_NB, fin), lambda i: (i, 0)),
            pl.BlockSpec((fin, 100), lambda i: (0, 0)),
            pl.BlockSpec((100,), lambda i: (0,)),
            pl.BlockSpec((100, dmid), lambda i: (0, 0)),
            pl.BlockSpec((dmid,), lambda i: (0,)),
            pl.BlockSpec((dmid, 100), lambda i: (0, 0)),
            pl.BlockSpec((100,), lambda i: (0,)),
            pl.BlockSpec((100, dout), lambda i: (0, 0)),
            pl.BlockSpec((dout,), lambda i: (0,)),
        ],
        out_specs=[
            pl.BlockSpec((_NB, dmid), lambda i: (i, 0)),
            pl.BlockSpec((_NB, dout), lambda i: (i, 0)),
        ],
        out_shape=[
            jax.ShapeDtypeStruct((N, dmid), _f32),
            jax.ShapeDtypeStruct((N, dout), _f32),
        ],
    )(x, w1, b1, w2, b2, w3, b3, w4, b4)


_BB = 2048


def _cls_body(tl_r, al_r, tr_r, ar_r, w1a_r, w1b_r, w1c_r, w1d_r, b1_r,
              g1_r, be1_r, w2_r, b2_r, g2_r, be2_r, w3_r, b3_r, o_r):
    s = 1.0 / jnp.sqrt(1.0 + 1e-5)
    h = (jnp.dot(tl_r[...], w1a_r[...], preferred_element_type=_f32)
         + jnp.dot(al_r[...], w1b_r[...], preferred_element_type=_f32)
         + jnp.dot(tr_r[...], w1c_r[...], preferred_element_type=_f32)
         + jnp.dot(ar_r[...], w1d_r[...], preferred_element_type=_f32)
         + b1_r[...])
    h = jnp.maximum(h, 0.0) * s * g1_r[...] + be1_r[...]
    h = jnp.dot(h, w2_r[...], preferred_element_type=_f32) + b2_r[...]
    h = jnp.maximum(h, 0.0) * s * g2_r[...] + be2_r[...]
    o_r[...] = jax.nn.sigmoid(
        jnp.dot(h, w3_r[...], preferred_element_type=_f32) + b3_r[...])


def _tc_cls(tl, al, tr, ar, p):
    w1 = p['Wd1']
    return pl.pallas_call(
        _cls_body,
        grid=(B // _BB,),
        in_specs=[pl.BlockSpec((_BB, 64), lambda i: (i, 0))] * 4
        + [pl.BlockSpec((64, 512), lambda i: (0, 0))] * 4
        + [pl.BlockSpec((512,), lambda i: (0,))] * 3
        + [pl.BlockSpec((512, 256), lambda i: (0, 0))]
        + [pl.BlockSpec((256,), lambda i: (0,))] * 3
        + [pl.BlockSpec((256, 1), lambda i: (0, 0)),
           pl.BlockSpec((1,), lambda i: (0,))],
        out_specs=pl.BlockSpec((_BB, 1), lambda i: (i, 0)),
        out_shape=jax.ShapeDtypeStruct((B, 1), _f32),
    )(tl, al, tr, ar, w1[0:64], w1[64:128], w1[128:192], w1[192:256],
      p['bd1'], p['g1'], p['be1'], p['Wd2'], p['bd2'], p['g2'], p['be2'],
      p['Wd3'], p['bd3'])


def _alpha_mats(a_src, a_dst):
    a_s = jnp.zeros((128, 2), _f32).at[0:64, 0].set(a_src[0]).at[64:128, 1].set(a_src[1])
    a_d = jnp.zeros((128, 2), _f32).at[0:64, 0].set(a_dst[0]).at[64:128, 1].set(a_dst[1])
    return a_s, a_d


def _gat_layer(x, srcp, dstp, w, a_src, a_dst, bias, z2, z16):
    a_s, a_d = _alpha_mats(a_src, a_dst)
    fcs, asrc, adst = _tc_feat(x, w, a_s, a_d)
    den, acc = _sc_gat(srcp, dstp, asrc.reshape(-1), adst.reshape(-1),
                       fcs, z2, z16)
    return _tc_epi(acc, den.reshape(2, N, 2), fcs, asrc, adst, bias)


def kernel(edge_index, attr_mtx, x_pairs, p):
    pad = jnp.zeros((EP - E,), _i32)
    srcp = jnp.concatenate([edge_index[0], pad])
    dstp = jnp.concatenate([edge_index[1], pad])
    xl = x_pairs[:, 0]
    xr = x_pairs[:, 1]
    z2 = jnp.zeros((2 * N,), _f32)
    z16 = jnp.zeros((N, 16), _f32)

    h1 = _gat_layer(p['X'], srcp, dstp, p['W1'], p['as1'], p['ad1'],
                    p['b1'], z2, z16)
    gcn_out = _gat_layer(h1, srcp, dstp, p['W2'], p['as2'], p['ad2'],
                         p['b2'], z2, z16)

    attr_emb, a2t = _tc_mlp2(attr_mtx, p['Wa1'], p['ba1'], p['Wa2'],
                             p['ba2'], p['Wat1'], p['bat1'], p['Wat2'],
                             p['bat2'], 64, 64)
    topo, t2a = _tc_mlp2(gcn_out, p['Wt1'], p['bt1'], p['Wt2'], p['bt2'],
                         p['Wta1'], p['bta1'], p['Wta2'], p['bta2'], 64, 128)

    tl, al, tr, ar = _sc_pair(topo, attr_emb, xl, xr)
    out = _tc_cls(tl, al, tr, ar, p)
    return (out, gcn_out, t2a, a2t)


# speedup vs baseline: 33.2924x; 1.3205x over previous
import functools

import jax
import jax.numpy as jnp
from jax import lax
from jax.experimental import pallas as pl
from jax.experimental.pallas import tpu as pltpu
from jax.experimental.pallas import tpu_sc as plsc

N = 50000
E = 800000
B = 16384
EP = 819200
NW = 32
EW = EP // NW
BK = 1280
NBLK = EW // BK
RPT = EW // 128
RPB = BK // 128

_f32 = jnp.float32
_i32 = jnp.int32


def _sc_gat_body(src_h, dst_h, asrc_h, adst_h, f0_h, f1_h, f2_h, f3_h,
                 f4_h, f5_h, f6_h, f7_h, z2_h, z16_h,
                 den_o, acc_o, ex_o,
                 den_sp, acc_sp, src_i, dst_i, vsrc_i, vdst_i, sg, dg,
                 exblk, rows, src_i2, dst_i2, exblk2, rows2,
                 semg0, semg1, semsc0, semsc1):
    c = lax.axis_index("c")
    s = lax.axis_index("s")
    wid = s * 2 + c
    lane = jnp.arange(16, dtype=_i32)
    half = lane >> 1
    par = lane & 1

    @pl.when(s == 0)
    def _():
        pltpu.sync_copy(z2_h, den_sp)
    plsc.subcore_barrier()

    def blk_a(blk, carry):
        ebase = wid * EW + blk * BK
        pltpu.sync_copy(src_h.at[pl.ds(ebase, BK)], src_i)
        pltpu.sync_copy(dst_h.at[pl.ds(ebase, BK)], dst_i)

        def mkidx(i, carry2):
            eloc = i * 8 + half
            vsrc_i[pl.ds(i * 16, 16)] = (
                plsc.load_gather(src_i, [eloc]) * 2 + par)
            vdst_i[pl.ds(i * 16, 16)] = (
                plsc.load_gather(dst_i, [eloc]) * 2 + par)
            return carry2
        lax.fori_loop(0, 2 * BK // 16, mkidx, 0)
        pltpu.sync_copy(asrc_h.at[vsrc_i], sg)
        pltpu.sync_copy(adst_h.at[vdst_i], dg)

        def cmp16(i, carry2):
            sl = pl.ds(i * 16, 16)
            a = sg[sl] + dg[sl]
            a = jnp.where(a > 0, a, 0.2 * a)
            ev = jnp.exp(a)
            ev = jnp.where(ebase + i * 8 + half < E, ev, 0.0)
            exblk[sl] = ev
            return carry2
        lax.fori_loop(0, 2 * BK // 16, cmp16, 0)
        pltpu.sync_copy(exblk, den_sp.at[vdst_i], add=True)
        pltpu.sync_copy(exblk, ex_o.at[pl.ds(2 * ebase, 2 * BK)])
        return carry
    lax.fori_loop(0, NBLK, blk_a, 0)

    plsc.subcore_barrier()

    @pl.when(s == 0)
    def _():
        pltpu.sync_copy(den_sp, den_o.at[c])

    src_b = (src_i, src_i2)
    dst_b = (dst_i, dst_i2)
    exb_b = (exblk, exblk2)
    rows_b = (rows, rows2)
    semg = (semg0, semg1)
    semsc = (semsc0, semsc1)

    for chunk in range(8):
        h = chunk // 4
        fc_h = (f0_h, f1_h, f2_h, f3_h, f4_h, f5_h, f6_h, f7_h)[chunk]

        @pl.when(s == 0)
        def _():
            pltpu.sync_copy(z16_h, acc_sp)
        plsc.subcore_barrier()

        def load_blk(blk, bi, fc_h=fc_h):
            ebase = wid * EW + blk * BK
            pltpu.sync_copy(src_h.at[pl.ds(ebase, BK)], src_b[bi])
            pltpu.sync_copy(dst_h.at[pl.ds(ebase, BK)], dst_b[bi])
            pltpu.sync_copy(ex_o.at[pl.ds(2 * ebase, 2 * BK)], exb_b[bi])
            pltpu.async_copy(fc_h.at[src_b[bi]], rows_b[bi], semg[bi])

        def proc(blk, bi, fc_h=fc_h, h=h):
            nbi = 1 - bi

            @pl.when(blk + 1 < NBLK)
            def _():
                @pl.when(blk >= 1)
                def _():
                    pltpu.make_async_copy(
                        rows_b[nbi], acc_sp.at[dst_b[nbi]],
                        semsc[nbi]).wait()
                load_blk(blk + 1, nbi)

            pltpu.make_async_copy(fc_h.at[src_b[bi]], rows_b[bi],
                                  semg[bi]).wait()
            rr = rows_b[bi]
            ee = exb_b[bi]

            def edge8(j, carry2):
                e0 = j * 8
                for u in range(8):
                    e = e0 + u
                    exv = plsc.load_gather(
                        ee, [jnp.full((16,), 2 * e + h, dtype=_i32)])
                    rr[e, :] = rr[e, :] * exv
                return carry2
            lax.fori_loop(0, BK // 8, edge8, 0)
            pltpu.async_copy(rows_b[bi], acc_sp.at[dst_b[bi]], semsc[bi],
                             add=True)

        load_blk(0, 0)

        def pair(j, carry):
            proc(2 * j, 0)
            proc(2 * j + 1, 1)
            return carry
        lax.fori_loop(0, NBLK // 2, pair, 0)
        pltpu.make_async_copy(rows_b[0], acc_sp.at[dst_b[0]], semsc[0]).wait()
        pltpu.make_async_copy(rows_b[1], acc_sp.at[dst_b[1]], semsc[1]).wait()

        plsc.subcore_barrier()

        @pl.when(s == 0)
        def _(chunk=chunk):
            pltpu.sync_copy(acc_sp, acc_o.at[c * 8 + chunk])


def _sc_gat(srcp, dstp, asrc, adst, fcs, z2, z16):
    fn = pl.kernel(
        _sc_gat_body,
        out_type=(jax.ShapeDtypeStruct((2, 2 * N), _f32),
                  jax.ShapeDtypeStruct((16, N, 16), _f32),
                  jax.ShapeDtypeStruct((2 * EP,), _f32)),
        mesh=plsc.VectorSubcoreMesh(core_axis_name="c", subcore_axis_name="s"),
        scratch_types=[
            pltpu.VMEM_SHARED((2 * N,), _f32),
            pltpu.VMEM_SHARED((N, 16), _f32),
            pltpu.VMEM((BK,), _i32),
            pltpu.VMEM((BK,), _i32),
            pltpu.VMEM((2 * BK,), _i32),
            pltpu.VMEM((2 * BK,), _i32),
            pltpu.VMEM((2 * BK,), _f32),
            pltpu.VMEM((2 * BK,), _f32),
            pltpu.VMEM((2 * BK,), _f32),
            pltpu.VMEM((BK, 16), _f32),
            pltpu.VMEM((BK,), _i32),
            pltpu.VMEM((BK,), _i32),
            pltpu.VMEM((2 * BK,), _f32),
            pltpu.VMEM((BK, 16), _f32),
            pltpu.SemaphoreType.DMA,
            pltpu.SemaphoreType.DMA,
            pltpu.SemaphoreType.DMA,
            pltpu.SemaphoreType.DMA,
        ],
        compiler_params=pltpu.CompilerParams(needs_layout_passes=False,
                                             use_tc_tiling_on_sc=False),
    )
    den, acc, _ex = fn(srcp, dstp, asrc, adst, *fcs, z2, z16)
    return den, acc


def _sc_pair_body(topo_h, attr_h, xl_h, xr_h, tl_o, al_o, tr_o, ar_o,
                  pidx, prow):
    c = lax.axis_index("c")
    s = lax.axis_index("s")
    wid = s * 2 + c
    base = wid * (B // NW)
    pltpu.sync_copy(xl_h.at[pl.ds(base, B // NW)], pidx)
    pltpu.sync_copy(topo_h.at[pidx], prow)
    pltpu.sync_copy(prow, tl_o.at[pl.ds(base, B // NW)])
    pltpu.sync_copy(attr_h.at[pidx], prow)
    pltpu.sync_copy(prow, al_o.at[pl.ds(base, B // NW)])
    pltpu.sync_copy(xr_h.at[pl.ds(base, B // NW)], pidx)
    pltpu.sync_copy(topo_h.at[pidx], prow)
    pltpu.sync_copy(prow, tr_o.at[pl.ds(base, B // NW)])
    pltpu.sync_copy(attr_h.at[pidx], prow)
    pltpu.sync_copy(prow, ar_o.at[pl.ds(base, B // NW)])


def _sc_pair(topo, attr, xl, xr):
    fn = pl.kernel(
        _sc_pair_body,
        out_type=tuple(jax.ShapeDtypeStruct((B, 64), _f32) for _ in range(4)),
        mesh=plsc.VectorSubcoreMesh(core_axis_name="c", subcore_axis_name="s"),
        scratch_types=[
            pltpu.VMEM((B // NW,), _i32),
            pltpu.VMEM((B // NW, 64), _f32),
        ],
        compiler_params=pltpu.CompilerParams(needs_layout_passes=False,
                                             use_tc_tiling_on_sc=False),
    )
    return fn(topo, attr, xl, xr)


_NB = 2000


def _feat_body(x_r, w_r, as_r, ad_r, *rest):
    f_rs = rest[0:8]
    s_r, d_r = rest[8], rest[9]
    f = jnp.dot(x_r[...], w_r[...], preferred_element_type=_f32)
    for g in range(8):
        f_rs[g][...] = f[:, 16 * g:16 * (g + 1)]
    s_r[...] = jnp.dot(f, as_r[...], preferred_element_type=_f32)
    d_r[...] = jnp.dot(f, ad_r[...], preferred_element_type=_f32)


def _tc_feat(x, w, a_s, a_d):
    fin = x.shape[1]
    outs = pl.pallas_call(
        _feat_body,
        grid=(N // _NB,),
        in_specs=[
            pl.BlockSpec((_NB, fin), lambda i: (i, 0)),
            pl.BlockSpec((fin, 128), lambda i: (0, 0)),
            pl.BlockSpec((128, 2), lambda i: (0, 0)),
            pl.BlockSpec((128, 2), lambda i: (0, 0)),
        ],
        out_specs=[pl.BlockSpec((_NB, 16), lambda i: (i, 0))] * 8
        + [pl.BlockSpec((_NB, 2), lambda i: (i, 0))] * 2,
        out_shape=[jax.ShapeDtypeStruct((N, 16), _f32)] * 8
        + [jax.ShapeDtypeStruct((N, 2), _f32)] * 2,
    )(x, w, a_s, a_d)
    return outs[0:8], outs[8], outs[9]


def _epi_body(acc_r, den_r, *rest):
    f_rs = rest[0:8]
    s_r, d_r, b_r, o_r = rest[8], rest[9], rest[10], rest[11]
    a = s_r[...] + d_r[...]
    exs = jnp.exp(jnp.where(a > 0, a, 0.2 * a))
    den = den_r[0] + den_r[1] + exs + 1e-16
    for q in range(4):
        n0 = acc_r[q] + acc_r[8 + q] + f_rs[q][...] * exs[:, 0:1]
        n1 = acc_r[4 + q] + acc_r[12 + q] + f_rs[4 + q][...] * exs[:, 1:2]
        o_r[:, q * 16:(q + 1) * 16] = (
            0.5 * (n0 / den[:, 0:1] + n1 / den[:, 1:2])
            + b_r[q * 16:(q + 1) * 16])


def _tc_epi(acc, den, fcs, asrc, adst, bias):
    nb = 1000
    return pl.pallas_call(
        _epi_body,
        grid=(N // nb,),
        in_specs=[
            pl.BlockSpec((16, nb, 16), lambda i: (0, i, 0)),
            pl.BlockSpec((2, nb, 2), lambda i: (0, i, 0)),
        ]
        + [pl.BlockSpec((nb, 16), lambda i: (i, 0))] * 8
        + [pl.BlockSpec((nb, 2), lambda i: (i, 0))] * 2
        + [pl.BlockSpec((64,), lambda i: (0,))],
        out_specs=pl.BlockSpec((nb, 64), lambda i: (i, 0)),
        out_shape=jax.ShapeDtypeStruct((N, 64), _f32),
    )(acc, den, *fcs, asrc, adst, bias)


def _mlp2_body(x_r, w1_r, b1_r, w2_r, b2_r, w3_r, b3_r, w4_r, b4_r,
               y2_r, y4_r):
    t = jnp.dot(x_r[...], w1_r[...], preferred_element_type=_f32) + b1_r[...]
    y2 = jnp.dot(t, w2_r[...], preferred_element_type=_f32) + b2_r[...]
    y2_r[...] = y2
    u = jnp.dot(y2, w3_r[...], preferred_element_type=_f32) + b3_r[...]
    y4_r[...] = jnp.dot(u, w4_r[...], preferred_element_type=_f32) + b4_r[...]


def _tc_mlp2(x, w1, b1, w2, b2, w3, b3, w4, b4, dmid, dout):
    fin = x.shape[1]
    return pl.pallas_call(
        _mlp2_body,
        grid=(N // _NB,),
        in_specs=[
            pl.BlockSpec((_NB, fin), lambda i: (i, 0)),
            pl.BlockSpec((fin, 100), lambda i: (0, 0)),
            pl.BlockSpec((100,), lambda i: (0,)),
            pl.BlockSpec((100, dmid), lambda i: (0, 0)),
            pl.BlockSpec((dmid,), lambda i: (0,)),
            pl.BlockSpec((dmid, 100), lambda i: (0, 0)),
            pl.BlockSpec((100,), lambda i: (0,)),
            pl.BlockSpec((100, dout), lambda i: (0, 0)),
            pl.BlockSpec((dout,), lambda i: (0,)),
        ],
        out_specs=[
            pl.BlockSpec((_NB, dmid), lambda i: (i, 0)),
            pl.BlockSpec((_NB, dout), lambda i: (i, 0)),
        ],
        out_shape=[
            jax.ShapeDtypeStruct((N, dmid), _f32),
            jax.ShapeDtypeStruct((N, dout), _f32),
        ],
    )(x, w1, b1, w2, b2, w3, b3, w4, b4)


_BB = 2048


def _cls_body(tl_r, al_r, tr_r, ar_r, w1a_r, w1b_r, w1c_r, w1d_r, b1_r,
              g1_r, be1_r, w2_r, b2_r, g2_r, be2_r, w3_r, b3_r, o_r):
    s = 1.0 / jnp.sqrt(1.0 + 1e-5)
    h = (jnp.dot(tl_r[...], w1a_r[...], preferred_element_type=_f32)
         + jnp.dot(al_r[...], w1b_r[...], preferred_element_type=_f32)
         + jnp.dot(tr_r[...], w1c_r[...], preferred_element_type=_f32)
         + jnp.dot(ar_r[...], w1d_r[...], preferred_element_type=_f32)
         + b1_r[...])
    h = jnp.maximum(h, 0.0) * s * g1_r[...] + be1_r[...]
    h = jnp.dot(h, w2_r[...], preferred_element_type=_f32) + b2_r[...]
    h = jnp.maximum(h, 0.0) * s * g2_r[...] + be2_r[...]
    o_r[...] = jax.nn.sigmoid(
        jnp.dot(h, w3_r[...], preferred_element_type=_f32) + b3_r[...])


def _tc_cls(tl, al, tr, ar, p):
    w1 = p['Wd1']
    return pl.pallas_call(
        _cls_body,
        grid=(B // _BB,),
        in_specs=[pl.BlockSpec((_BB, 64), lambda i: (i, 0))] * 4
        + [pl.BlockSpec((64, 512), lambda i: (0, 0))] * 4
        + [pl.BlockSpec((512,), lambda i: (0,))] * 3
        + [pl.BlockSpec((512, 256), lambda i: (0, 0))]
        + [pl.BlockSpec((256,), lambda i: (0,))] * 3
        + [pl.BlockSpec((256, 1), lambda i: (0, 0)),
           pl.BlockSpec((1,), lambda i: (0,))],
        out_specs=pl.BlockSpec((_BB, 1), lambda i: (i, 0)),
        out_shape=jax.ShapeDtypeStruct((B, 1), _f32),
    )(tl, al, tr, ar, w1[0:64], w1[64:128], w1[128:192], w1[192:256],
      p['bd1'], p['g1'], p['be1'], p['Wd2'], p['bd2'], p['g2'], p['be2'],
      p['Wd3'], p['bd3'])


def _alpha_mats(a_src, a_dst):
    a_s = jnp.zeros((128, 2), _f32).at[0:64, 0].set(a_src[0]).at[64:128, 1].set(a_src[1])
    a_d = jnp.zeros((128, 2), _f32).at[0:64, 0].set(a_dst[0]).at[64:128, 1].set(a_dst[1])
    return a_s, a_d


def _gat_layer(x, srcp, dstp, w, a_src, a_dst, bias, z2, z16):
    a_s, a_d = _alpha_mats(a_src, a_dst)
    fcs, asrc, adst = _tc_feat(x, w, a_s, a_d)
    den, acc = _sc_gat(srcp, dstp, asrc.reshape(-1), adst.reshape(-1),
                       fcs, z2, z16)
    return _tc_epi(acc, den.reshape(2, N, 2), fcs, asrc, adst, bias)


def kernel(edge_index, attr_mtx, x_pairs, p):
    pad = jnp.zeros((EP - E,), _i32)
    srcp = jnp.concatenate([edge_index[0], pad])
    dstp = jnp.concatenate([edge_index[1], pad])
    xl = x_pairs[:, 0]
    xr = x_pairs[:, 1]
    z2 = jnp.zeros((2 * N,), _f32)
    z16 = jnp.zeros((N, 16), _f32)

    h1 = _gat_layer(p['X'], srcp, dstp, p['W1'], p['as1'], p['ad1'],
                    p['b1'], z2, z16)
    gcn_out = _gat_layer(h1, srcp, dstp, p['W2'], p['as2'], p['ad2'],
                         p['b2'], z2, z16)

    attr_emb, a2t = _tc_mlp2(attr_mtx, p['Wa1'], p['ba1'], p['Wa2'],
                             p['ba2'], p['Wat1'], p['bat1'], p['Wat2'],
                             p['bat2'], 64, 64)
    topo, t2a = _tc_mlp2(gcn_out, p['Wt1'], p['bt1'], p['Wt2'], p['bt2'],
                         p['Wta1'], p['bta1'], p['Wta2'], p['bta2'], 64, 128)

    tl, al, tr, ar = _sc_pair(topo, attr_emb, xl, xr)
    out = _tc_cls(tl, al, tr, ar, p)
    return (out, gcn_out, t2a, a2t)


# speedup vs baseline: 34.1265x; 1.0251x over previous
import functools

import jax
import jax.numpy as jnp
from jax import lax
from jax.experimental import pallas as pl
from jax.experimental.pallas import tpu as pltpu
from jax.experimental.pallas import tpu_sc as plsc

N = 50000
E = 800000
B = 16384
EP = 819200
NW = 32
EW = EP // NW
BK = 800
NBLK = EW // BK
RPT = EW // 128
RPB = BK // 128

_f32 = jnp.float32
_i32 = jnp.int32


def _sc_gat_body(src_h, dst_h, asrc_h, adst_h, f0_h, f1_h, f2_h, f3_h,
                 f4_h, f5_h, f6_h, f7_h, z2_h, z16_h,
                 den_o, acc_o, ex_o,
                 den_sp, acc_sp, src_i, dst_i, vsrc_i, vdst_i, sg, dg,
                 exblk, rows, src_i2, dst_i2, exblk2, rows2,
                 vsrc_i2, vdst_i2, sg2, dg2,
                 semg0, semg1, semsc0, semsc1,
                 sema0, sema1, semd0, semd1, seme0, seme1):
    c = lax.axis_index("c")
    s = lax.axis_index("s")
    wid = s * 2 + c
    lane = jnp.arange(16, dtype=_i32)
    half = lane >> 1
    par = lane & 1

    src_b = (src_i, src_i2)
    dst_b = (dst_i, dst_i2)
    vsrc_b = (vsrc_i, vsrc_i2)
    vdst_b = (vdst_i, vdst_i2)
    sg_b = (sg, sg2)
    dg_b = (dg, dg2)
    exb_b = (exblk, exblk2)
    sema = (sema0, sema1)
    semd = (semd0, semd1)
    seme = (seme0, seme1)

    @pl.when(s == 0)
    def _():
        pltpu.sync_copy(z2_h, den_sp)
    plsc.subcore_barrier()

    def load_a(blk, bi):
        ebase = wid * EW + blk * BK
        pltpu.sync_copy(src_h.at[pl.ds(ebase, BK)], src_b[bi])
        pltpu.sync_copy(dst_h.at[pl.ds(ebase, BK)], dst_b[bi])

        def mkidx(i, carry2):
            eloc = i * 8 + half
            vsrc_b[bi][pl.ds(i * 16, 16)] = (
                plsc.load_gather(src_b[bi], [eloc]) * 2 + par)
            vdst_b[bi][pl.ds(i * 16, 16)] = (
                plsc.load_gather(dst_b[bi], [eloc]) * 2 + par)
            return carry2
        lax.fori_loop(0, 2 * BK // 16, mkidx, 0)
        pltpu.async_copy(asrc_h.at[vsrc_b[bi]], sg_b[bi], sema[bi])
        pltpu.async_copy(adst_h.at[vdst_b[bi]], dg_b[bi], sema[bi])

    def proc_a(blk, bi):
        nbi = 1 - bi

        @pl.when(blk + 1 < NBLK)
        def _():
            @pl.when(blk >= 1)
            def _():
                pltpu.make_async_copy(
                    exb_b[nbi], den_sp.at[vdst_b[nbi]], semd[nbi]).wait()
                pltpu.make_async_copy(
                    exb_b[nbi], ex_o.at[pl.ds(0, 2 * BK)], seme[nbi]).wait()
            load_a(blk + 1, nbi)

        ebase = wid * EW + blk * BK
        pltpu.make_async_copy(asrc_h.at[vsrc_b[bi]], sg_b[bi],
                              sema[bi]).wait()
        pltpu.make_async_copy(adst_h.at[vdst_b[bi]], dg_b[bi],
                              sema[bi]).wait()

        def cmp16(i, carry2):
            sl = pl.ds(i * 16, 16)
            a = sg_b[bi][sl] + dg_b[bi][sl]
            a = jnp.where(a > 0, a, 0.2 * a)
            ev = jnp.exp(a)
            ev = jnp.where(ebase + i * 8 + half < E, ev, 0.0)
            exb_b[bi][sl] = ev
            return carry2
        lax.fori_loop(0, 2 * BK // 16, cmp16, 0)
        pltpu.async_copy(exb_b[bi], den_sp.at[vdst_b[bi]], semd[bi],
                         add=True)
        pltpu.async_copy(exb_b[bi], ex_o.at[pl.ds(2 * ebase, 2 * BK)],
                         seme[bi])

    load_a(0, 0)

    def pair_a(j, carry):
        proc_a(2 * j, 0)
        proc_a(2 * j + 1, 1)
        return carry
    lax.fori_loop(0, NBLK // 2, pair_a, 0)
    for bi in range(2):
        pltpu.make_async_copy(exb_b[bi], den_sp.at[vdst_b[bi]],
                              semd[bi]).wait()
        pltpu.make_async_copy(exb_b[bi], ex_o.at[pl.ds(0, 2 * BK)],
                              seme[bi]).wait()

    plsc.subcore_barrier()

    @pl.when(s == 0)
    def _():
        pltpu.sync_copy(den_sp, den_o.at[c])

    rows_b = (rows, rows2)
    semg = (semg0, semg1)
    semsc = (semsc0, semsc1)

    for chunk in range(8):
        h = chunk // 4
        fc_h = (f0_h, f1_h, f2_h, f3_h, f4_h, f5_h, f6_h, f7_h)[chunk]

        @pl.when(s == 0)
        def _():
            pltpu.sync_copy(z16_h, acc_sp)
        plsc.subcore_barrier()

        def load_blk(blk, bi, fc_h=fc_h):
            ebase = wid * EW + blk * BK
            pltpu.sync_copy(src_h.at[pl.ds(ebase, BK)], src_b[bi])
            pltpu.sync_copy(dst_h.at[pl.ds(ebase, BK)], dst_b[bi])
            pltpu.sync_copy(ex_o.at[pl.ds(2 * ebase, 2 * BK)], exb_b[bi])
            pltpu.async_copy(fc_h.at[src_b[bi]], rows_b[bi], semg[bi])

        def proc(blk, bi, fc_h=fc_h, h=h):
            nbi = 1 - bi

            @pl.when(blk + 1 < NBLK)
            def _():
                @pl.when(blk >= 1)
                def _():
                    pltpu.make_async_copy(
                        rows_b[nbi], acc_sp.at[dst_b[nbi]],
                        semsc[nbi]).wait()
                load_blk(blk + 1, nbi)

            pltpu.make_async_copy(fc_h.at[src_b[bi]], rows_b[bi],
                                  semg[bi]).wait()
            rr = rows_b[bi]
            ee = exb_b[bi]

            def edge8(j, carry2):
                e0 = j * 8
                for u in range(8):
                    e = e0 + u
                    exv = plsc.load_gather(
                        ee, [jnp.full((16,), 2 * e + h, dtype=_i32)])
                    rr[e, :] = rr[e, :] * exv
                return carry2
            lax.fori_loop(0, BK // 8, edge8, 0)
            pltpu.async_copy(rows_b[bi], acc_sp.at[dst_b[bi]], semsc[bi],
                             add=True)

        load_blk(0, 0)

        def pair(j, carry):
            proc(2 * j, 0)
            proc(2 * j + 1, 1)
            return carry
        lax.fori_loop(0, NBLK // 2, pair, 0)
        pltpu.make_async_copy(rows_b[0], acc_sp.at[dst_b[0]], semsc[0]).wait()
        pltpu.make_async_copy(rows_b[1], acc_sp.at[dst_b[1]], semsc[1]).wait()

        plsc.subcore_barrier()

        @pl.when(s == 0)
        def _(chunk=chunk):
            pltpu.sync_copy(acc_sp, acc_o.at[c * 8 + chunk])


def _sc_gat(srcp, dstp, asrc, adst, fcs, z2, z16):
    fn = pl.kernel(
        _sc_gat_body,
        out_type=(jax.ShapeDtypeStruct((2, 2 * N), _f32),
                  jax.ShapeDtypeStruct((16, N, 16), _f32),
                  jax.ShapeDtypeStruct((2 * EP,), _f32)),
        mesh=plsc.VectorSubcoreMesh(core_axis_name="c", subcore_axis_name="s"),
        scratch_types=[
            pltpu.VMEM_SHARED((2 * N,), _f32),
            pltpu.VMEM_SHARED((N, 16), _f32),
            pltpu.VMEM((BK,), _i32),
            pltpu.VMEM((BK,), _i32),
            pltpu.VMEM((2 * BK,), _i32),
            pltpu.VMEM((2 * BK,), _i32),
            pltpu.VMEM((2 * BK,), _f32),
            pltpu.VMEM((2 * BK,), _f32),
            pltpu.VMEM((2 * BK,), _f32),
            pltpu.VMEM((BK, 16), _f32),
            pltpu.VMEM((BK,), _i32),
            pltpu.VMEM((BK,), _i32),
            pltpu.VMEM((2 * BK,), _f32),
            pltpu.VMEM((BK, 16), _f32),
            pltpu.VMEM((2 * BK,), _i32),
            pltpu.VMEM((2 * BK,), _i32),
            pltpu.VMEM((2 * BK,), _f32),
            pltpu.VMEM((2 * BK,), _f32),
        ] + [pltpu.SemaphoreType.DMA] * 10,
        compiler_params=pltpu.CompilerParams(needs_layout_passes=False,
                                             use_tc_tiling_on_sc=False),
    )
    den, acc, _ex = fn(srcp, dstp, asrc, adst, *fcs, z2, z16)
    return den, acc


def _sc_pair_body(topo_h, attr_h, xl_h, xr_h, tl_o, al_o, tr_o, ar_o,
                  pidx, prow):
    c = lax.axis_index("c")
    s = lax.axis_index("s")
    wid = s * 2 + c
    base = wid * (B // NW)
    pltpu.sync_copy(xl_h.at[pl.ds(base, B // NW)], pidx)
    pltpu.sync_copy(topo_h.at[pidx], prow)
    pltpu.sync_copy(prow, tl_o.at[pl.ds(base, B // NW)])
    pltpu.sync_copy(attr_h.at[pidx], prow)
    pltpu.sync_copy(prow, al_o.at[pl.ds(base, B // NW)])
    pltpu.sync_copy(xr_h.at[pl.ds(base, B // NW)], pidx)
    pltpu.sync_copy(topo_h.at[pidx], prow)
    pltpu.sync_copy(prow, tr_o.at[pl.ds(base, B // NW)])
    pltpu.sync_copy(attr_h.at[pidx], prow)
    pltpu.sync_copy(prow, ar_o.at[pl.ds(base, B // NW)])


def _sc_pair(topo, attr, xl, xr):
    fn = pl.kernel(
        _sc_pair_body,
        out_type=tuple(jax.ShapeDtypeStruct((B, 64), _f32) for _ in range(4)),
        mesh=plsc.VectorSubcoreMesh(core_axis_name="c", subcore_axis_name="s"),
        scratch_types=[
            pltpu.VMEM((B // NW,), _i32),
            pltpu.VMEM((B // NW, 64), _f32),
        ],
        compiler_params=pltpu.CompilerParams(needs_layout_passes=False,
                                             use_tc_tiling_on_sc=False),
    )
    return fn(topo, attr, xl, xr)


_NB = 2000


def _feat_body(x_r, w_r, as_r, ad_r, *rest):
    f_rs = rest[0:8]
    s_r, d_r = rest[8], rest[9]
    f = jnp.dot(x_r[...], w_r[...], preferred_element_type=_f32)
    for g in range(8):
        f_rs[g][...] = f[:, 16 * g:16 * (g + 1)]
    s_r[...] = jnp.dot(f, as_r[...], preferred_element_type=_f32)
    d_r[...] = jnp.dot(f, ad_r[...], preferred_element_type=_f32)


def _tc_feat(x, w, a_s, a_d):
    fin = x.shape[1]
    outs = pl.pallas_call(
        _feat_body,
        grid=(N // _NB,),
        in_specs=[
            pl.BlockSpec((_NB, fin), lambda i: (i, 0)),
            pl.BlockSpec((fin, 128), lambda i: (0, 0)),
            pl.BlockSpec((128, 2), lambda i: (0, 0)),
            pl.BlockSpec((128, 2), lambda i: (0, 0)),
        ],
        out_specs=[pl.BlockSpec((_NB, 16), lambda i: (i, 0))] * 8
        + [pl.BlockSpec((_NB, 2), lambda i: (i, 0))] * 2,
        out_shape=[jax.ShapeDtypeStruct((N, 16), _f32)] * 8
        + [jax.ShapeDtypeStruct((N, 2), _f32)] * 2,
    )(x, w, a_s, a_d)
    return outs[0:8], outs[8], outs[9]


def _epi_body(acc_r, den_r, *rest):
    f_rs = rest[0:8]
    s_r, d_r, b_r, o_r = rest[8], rest[9], rest[10], rest[11]
    a = s_r[...] + d_r[...]
    exs = jnp.exp(jnp.where(a > 0, a, 0.2 * a))
    den = den_r[0] + den_r[1] + exs + 1e-16
    for q in range(4):
        n0 = acc_r[q] + acc_r[8 + q] + f_rs[q][...] * exs[:, 0:1]
        n1 = acc_r[4 + q] + acc_r[12 + q] + f_rs[4 + q][...] * exs[:, 1:2]
        o_r[:, q * 16:(q + 1) * 16] = (
            0.5 * (n0 / den[:, 0:1] + n1 / den[:, 1:2])
            + b_r[q * 16:(q + 1) * 16])


def _tc_epi(acc, den, fcs, asrc, adst, bias):
    nb = 1000
    return pl.pallas_call(
        _epi_body,
        grid=(N // nb,),
        in_specs=[
            pl.BlockSpec((16, nb, 16), lambda i: (0, i, 0)),
            pl.BlockSpec((2, nb, 2), lambda i: (0, i, 0)),
        ]
        + [pl.BlockSpec((nb, 16), lambda i: (i, 0))] * 8
        + [pl.BlockSpec((nb, 2), lambda i: (i, 0))] * 2
        + [pl.BlockSpec((64,), lambda i: (0,))],
        out_specs=pl.BlockSpec((nb, 64), lambda i: (i, 0)),
        out_shape=jax.ShapeDtypeStruct((N, 64), _f32),
    )(acc, den, *fcs, asrc, adst, bias)


def _mlp2_body(x_r, w1_r, b1_r, w2_r, b2_r, w3_r, b3_r, w4_r, b4_r,
               y2_r, y4_r):
    t = jnp.dot(x_r[...], w1_r[...], preferred_element_type=_f32) + b1_r[...]
    y2 = jnp.dot(t, w2_r[...], preferred_element_type=_f32) + b2_r[...]
    y2_r[...] = y2
    u = jnp.dot(y2, w3_r[...], preferred_element_type=_f32) + b3_r[...]
    y4_r[...] = jnp.dot(u, w4_r[...], preferred_element_type=_f32) + b4_r[...]


def _tc_mlp2(x, w1, b1, w2, b2, w3, b3, w4, b4, dmid, dout):
    fin = x.shape[1]
    return pl.pallas_call(
        _mlp2_body,
        grid=(N // _NB,),
        in_specs=[
            pl.BlockSpec((_NB, fin), lambda i: (i, 0)),
            pl.BlockSpec((fin, 100), lambda i: (0, 0)),
            pl.BlockSpec((100,), lambda i: (0,)),
            pl.BlockSpec((100, dmid), lambda i: (0, 0)),
            pl.BlockSpec((dmid,), lambda i: (0,)),
            pl.BlockSpec((dmid, 100), lambda i: (0, 0)),
            pl.BlockSpec((100,), lambda i: (0,)),
            pl.BlockSpec((100, dout), lambda i: (0, 0)),
            pl.BlockSpec((dout,), lambda i: (0,)),
        ],
        out_specs=[
            pl.BlockSpec((_NB, dmid), lambda i: (i, 0)),
            pl.BlockSpec((_NB, dout), lambda i: (i, 0)),
        ],
        out_shape=[
            jax.ShapeDtypeStruct((N, dmid), _f32),
            jax.ShapeDtypeStruct((N, dout), _f32),
        ],
    )(x, w1, b1, w2, b2, w3, b3, w4, b4)


_BB = 2048


def _cls_body(tl_r, al_r, tr_r, ar_r, w1a_r, w1b_r, w1c_r, w1d_r, b1_r,
              g1_r, be1_r, w2_r, b2_r, g2_r, be2_r, w3_r, b3_r, o_r):
    s = 1.0 / jnp.sqrt(1.0 + 1e-5)
    h = (jnp.dot(tl_r[...], w1a_r[...], preferred_element_type=_f32)
         + jnp.dot(al_r[...], w1b_r[...], preferred_element_type=_f32)
         + jnp.dot(tr_r[...], w1c_r[...], preferred_element_type=_f32)
         + jnp.dot(ar_r[...], w1d_r[...], preferred_element_type=_f32)
         + b1_r[...])
    h = jnp.maximum(h, 0.0) * s * g1_r[...] + be1_r[...]
    h = jnp.dot(h, w2_r[...], preferred_element_type=_f32) + b2_r[...]
    h = jnp.maximum(h, 0.0) * s * g2_r[...] + be2_r[...]
    o_r[...] = jax.nn.sigmoid(
        jnp.dot(h, w3_r[...], preferred_element_type=_f32) + b3_r[...])


def _tc_cls(tl, al, tr, ar, p):
    w1 = p['Wd1']
    return pl.pallas_call(
        _cls_body,
        grid=(B // _BB,),
        in_specs=[pl.BlockSpec((_BB, 64), lambda i: (i, 0))] * 4
        + [pl.BlockSpec((64, 512), lambda i: (0, 0))] * 4
        + [pl.BlockSpec((512,), lambda i: (0,))] * 3
        + [pl.BlockSpec((512, 256), lambda i: (0, 0))]
        + [pl.BlockSpec((256,), lambda i: (0,))] * 3
        + [pl.BlockSpec((256, 1), lambda i: (0, 0)),
           pl.BlockSpec((1,), lambda i: (0,))],
        out_specs=pl.BlockSpec((_BB, 1), lambda i: (i, 0)),
        out_shape=jax.ShapeDtypeStruct((B, 1), _f32),
    )(tl, al, tr, ar, w1[0:64], w1[64:128], w1[128:192], w1[192:256],
      p['bd1'], p['g1'], p['be1'], p['Wd2'], p['bd2'], p['g2'], p['be2'],
      p['Wd3'], p['bd3'])


def _alpha_mats(a_src, a_dst):
    a_s = jnp.zeros((128, 2), _f32).at[0:64, 0].set(a_src[0]).at[64:128, 1].set(a_src[1])
    a_d = jnp.zeros((128, 2), _f32).at[0:64, 0].set(a_dst[0]).at[64:128, 1].set(a_dst[1])
    return a_s, a_d


def _gat_layer(x, srcp, dstp, w, a_src, a_dst, bias, z2, z16):
    a_s, a_d = _alpha_mats(a_src, a_dst)
    fcs, asrc, adst = _tc_feat(x, w, a_s, a_d)
    den, acc = _sc_gat(srcp, dstp, asrc.reshape(-1), adst.reshape(-1),
                       fcs, z2, z16)
    return _tc_epi(acc, den.reshape(2, N, 2), fcs, asrc, adst, bias)


def kernel(edge_index, attr_mtx, x_pairs, p):
    pad = jnp.zeros((EP - E,), _i32)
    srcp = jnp.concatenate([edge_index[0], pad])
    dstp = jnp.concatenate([edge_index[1], pad])
    xl = x_pairs[:, 0]
    xr = x_pairs[:, 1]
    z2 = jnp.zeros((2 * N,), _f32)
    z16 = jnp.zeros((N, 16), _f32)

    h1 = _gat_layer(p['X'], srcp, dstp, p['W1'], p['as1'], p['ad1'],
                    p['b1'], z2, z16)
    gcn_out = _gat_layer(h1, srcp, dstp, p['W2'], p['as2'], p['ad2'],
                         p['b2'], z2, z16)

    attr_emb, a2t = _tc_mlp2(attr_mtx, p['Wa1'], p['ba1'], p['Wa2'],
                             p['ba2'], p['Wat1'], p['bat1'], p['Wat2'],
                             p['bat2'], 64, 64)
    topo, t2a = _tc_mlp2(gcn_out, p['Wt1'], p['bt1'], p['Wt2'], p['bt2'],
                         p['Wta1'], p['bta1'], p['Wta2'], p['bta2'], 64, 128)

    tl, al, tr, ar = _sc_pair(topo, attr_emb, xl, xr)
    out = _tc_cls(tl, al, tr, ar, p)
    return (out, gcn_out, t2a, a2t)


# speedup vs baseline: 36.9416x; 1.0825x over previous
import functools

import jax
import jax.numpy as jnp
from jax import lax
from jax.experimental import pallas as pl
from jax.experimental.pallas import tpu as pltpu
from jax.experimental.pallas import tpu_sc as plsc

N = 50000
E = 800000
B = 16384
EP = 819200
NW = 32
EW = EP // NW
BK = 800
NBLK = EW // BK
RPT = EW // 128
RPB = BK // 128

_f32 = jnp.float32
_i32 = jnp.int32


def _sc_gat_body(src_h, dst_h, asrc_h, adst_h, f0_h, f1_h, f2_h, f3_h,
                 f4_h, f5_h, f6_h, f7_h, z2_h, z16_h,
                 den_o, acc_o, ex_o,
                 den_sp, acc_sp, src_i, dst_i, vsrc_i, vdst_i, sg, dg,
                 exblk, rows, src_i2, dst_i2, exblk2, rows2,
                 vsrc_i2, vdst_i2, sg2, dg2,
                 semg0, semg1, semsc0, semsc1,
                 sema0, sema1, semd0, semd1, seme0, seme1):
    c = lax.axis_index("c")
    s = lax.axis_index("s")
    wid = s * 2 + c
    lane = jnp.arange(16, dtype=_i32)
    half = lane >> 1
    par = lane & 1

    src_b = (src_i, src_i2)
    dst_b = (dst_i, dst_i2)
    vsrc_b = (vsrc_i, vsrc_i2)
    vdst_b = (vdst_i, vdst_i2)
    sg_b = (sg, sg2)
    dg_b = (dg, dg2)
    exb_b = (exblk, exblk2)
    sema = (sema0, sema1)
    semd = (semd0, semd1)
    seme = (seme0, seme1)

    @pl.when(s == 0)
    def _():
        pltpu.sync_copy(z2_h, den_sp)
    plsc.subcore_barrier()

    def load_a(blk, bi):
        ebase = wid * EW + blk * BK
        pltpu.sync_copy(src_h.at[pl.ds(ebase, BK)], src_b[bi])
        pltpu.sync_copy(dst_h.at[pl.ds(ebase, BK)], dst_b[bi])

        def mkidx(i, carry2):
            eloc = i * 8 + half
            vsrc_b[bi][pl.ds(i * 16, 16)] = (
                plsc.load_gather(src_b[bi], [eloc]) * 2 + par)
            vdst_b[bi][pl.ds(i * 16, 16)] = (
                plsc.load_gather(dst_b[bi], [eloc]) * 2 + par)
            return carry2
        lax.fori_loop(0, 2 * BK // 16, mkidx, 0)
        pltpu.async_copy(asrc_h.at[vsrc_b[bi]], sg_b[bi], sema[bi])
        pltpu.async_copy(adst_h.at[vdst_b[bi]], dg_b[bi], sema[bi])

    def proc_a(blk, bi):
        nbi = 1 - bi

        @pl.when(blk + 1 < NBLK)
        def _():
            @pl.when(blk >= 1)
            def _():
                pltpu.make_async_copy(
                    exb_b[nbi], den_sp.at[vdst_b[nbi]], semd[nbi]).wait()
                pltpu.make_async_copy(
                    exb_b[nbi], ex_o.at[pl.ds(0, 2 * BK)], seme[nbi]).wait()
            load_a(blk + 1, nbi)

        ebase = wid * EW + blk * BK
        pltpu.make_async_copy(asrc_h.at[vsrc_b[bi]], sg_b[bi],
                              sema[bi]).wait()
        pltpu.make_async_copy(adst_h.at[vdst_b[bi]], dg_b[bi],
                              sema[bi]).wait()

        def cmp16(i, carry2):
            sl = pl.ds(i * 16, 16)
            a = sg_b[bi][sl] + dg_b[bi][sl]
            a = jnp.where(a > 0, a, 0.2 * a)
            ev = jnp.exp(a)
            ev = jnp.where(ebase + i * 8 + half < E, ev, 0.0)
            exb_b[bi][sl] = ev
            return carry2
        lax.fori_loop(0, 2 * BK // 16, cmp16, 0)
        pltpu.async_copy(exb_b[bi], den_sp.at[vdst_b[bi]], semd[bi],
                         add=True)
        pltpu.async_copy(exb_b[bi], ex_o.at[pl.ds(2 * ebase, 2 * BK)],
                         seme[bi])

    load_a(0, 0)

    def pair_a(j, carry):
        proc_a(2 * j, 0)
        proc_a(2 * j + 1, 1)
        return carry
    lax.fori_loop(0, NBLK // 2, pair_a, 0)
    for bi in range(2):
        pltpu.make_async_copy(exb_b[bi], den_sp.at[vdst_b[bi]],
                              semd[bi]).wait()
        pltpu.make_async_copy(exb_b[bi], ex_o.at[pl.ds(0, 2 * BK)],
                              seme[bi]).wait()

    plsc.subcore_barrier()

    @pl.when(s == 0)
    def _():
        pltpu.sync_copy(den_sp, den_o.at[c])

    rows_b = (rows, rows2)
    semg = (semg0, semg1)
    semsc = (semsc0, semsc1)
    semi = (sema0, sema1)

    for chunk in range(8):
        h = chunk // 4
        fc_h = (f0_h, f1_h, f2_h, f3_h, f4_h, f5_h, f6_h, f7_h)[chunk]

        @pl.when(s == 0)
        def _():
            pltpu.sync_copy(z16_h, acc_sp)
        plsc.subcore_barrier()

        def load_blk(blk, bi, fc_h=fc_h):
            ebase = wid * EW + blk * BK
            pltpu.async_copy(src_h.at[pl.ds(ebase, BK)], src_b[bi],
                             semi[bi])
            pltpu.make_async_copy(src_h.at[pl.ds(ebase, BK)], src_b[bi],
                                  semi[bi]).wait()
            pltpu.async_copy(fc_h.at[src_b[bi]], rows_b[bi], semg[bi])
            pltpu.async_copy(dst_h.at[pl.ds(ebase, BK)], dst_b[bi],
                             semi[bi])
            pltpu.async_copy(ex_o.at[pl.ds(2 * ebase, 2 * BK)], exb_b[bi],
                             semi[bi])

        def proc(blk, bi, fc_h=fc_h, h=h):
            nbi = 1 - bi

            @pl.when(blk + 1 < NBLK)
            def _():
                @pl.when(blk >= 1)
                def _():
                    pltpu.make_async_copy(
                        rows_b[nbi], acc_sp.at[dst_b[nbi]],
                        semsc[nbi]).wait()
                load_blk(blk + 1, nbi)

            pltpu.make_async_copy(dst_h.at[pl.ds(0, BK)], dst_b[bi],
                                  semi[bi]).wait()
            pltpu.make_async_copy(ex_o.at[pl.ds(0, 2 * BK)], exb_b[bi],
                                  semi[bi]).wait()
            pltpu.make_async_copy(fc_h.at[src_b[bi]], rows_b[bi],
                                  semg[bi]).wait()
            rr = rows_b[bi]
            ee = exb_b[bi]

            def edge8(j, carry2):
                e0 = j * 8
                for u in range(8):
                    e = e0 + u
                    exv = plsc.load_gather(
                        ee, [jnp.full((16,), 2 * e + h, dtype=_i32)])
                    rr[e, :] = rr[e, :] * exv
                return carry2
            lax.fori_loop(0, BK // 8, edge8, 0)
            pltpu.async_copy(rows_b[bi], acc_sp.at[dst_b[bi]], semsc[bi],
                             add=True)

        load_blk(0, 0)

        def pair(j, carry):
            proc(2 * j, 0)
            proc(2 * j + 1, 1)
            return carry
        lax.fori_loop(0, NBLK // 2, pair, 0)
        pltpu.make_async_copy(rows_b[0], acc_sp.at[dst_b[0]], semsc[0]).wait()
        pltpu.make_async_copy(rows_b[1], acc_sp.at[dst_b[1]], semsc[1]).wait()

        plsc.subcore_barrier()

        @pl.when(s == 0)
        def _(chunk=chunk):
            pltpu.sync_copy(acc_sp, acc_o.at[c * 8 + chunk])


def _sc_gat(srcp, dstp, asrc, adst, fcs, z2, z16):
    fn = pl.kernel(
        _sc_gat_body,
        out_type=(jax.ShapeDtypeStruct((2, 2 * N), _f32),
                  jax.ShapeDtypeStruct((16, N, 16), _f32),
                  jax.ShapeDtypeStruct((2 * EP,), _f32)),
        mesh=plsc.VectorSubcoreMesh(core_axis_name="c", subcore_axis_name="s"),
        scratch_types=[
            pltpu.VMEM_SHARED((2 * N,), _f32),
            pltpu.VMEM_SHARED((N, 16), _f32),
            pltpu.VMEM((BK,), _i32),
            pltpu.VMEM((BK,), _i32),
            pltpu.VMEM((2 * BK,), _i32),
            pltpu.VMEM((2 * BK,), _i32),
            pltpu.VMEM((2 * BK,), _f32),
            pltpu.VMEM((2 * BK,), _f32),
            pltpu.VMEM((2 * BK,), _f32),
            pltpu.VMEM((BK, 16), _f32),
            pltpu.VMEM((BK,), _i32),
            pltpu.VMEM((BK,), _i32),
            pltpu.VMEM((2 * BK,), _f32),
            pltpu.VMEM((BK, 16), _f32),
            pltpu.VMEM((2 * BK,), _i32),
            pltpu.VMEM((2 * BK,), _i32),
            pltpu.VMEM((2 * BK,), _f32),
            pltpu.VMEM((2 * BK,), _f32),
        ] + [pltpu.SemaphoreType.DMA] * 10,
        compiler_params=pltpu.CompilerParams(needs_layout_passes=False,
                                             use_tc_tiling_on_sc=False),
    )
    den, acc, _ex = fn(srcp, dstp, asrc, adst, *fcs, z2, z16)
    return den, acc


def _sc_pair_body(topo_h, attr_h, xl_h, xr_h, tl_o, al_o, tr_o, ar_o,
                  pidx, prow):
    c = lax.axis_index("c")
    s = lax.axis_index("s")
    wid = s * 2 + c
    base = wid * (B // NW)
    pltpu.sync_copy(xl_h.at[pl.ds(base, B // NW)], pidx)
    pltpu.sync_copy(topo_h.at[pidx], prow)
    pltpu.sync_copy(prow, tl_o.at[pl.ds(base, B // NW)])
    pltpu.sync_copy(attr_h.at[pidx], prow)
    pltpu.sync_copy(prow, al_o.at[pl.ds(base, B // NW)])
    pltpu.sync_copy(xr_h.at[pl.ds(base, B // NW)], pidx)
    pltpu.sync_copy(topo_h.at[pidx], prow)
    pltpu.sync_copy(prow, tr_o.at[pl.ds(base, B // NW)])
    pltpu.sync_copy(attr_h.at[pidx], prow)
    pltpu.sync_copy(prow, ar_o.at[pl.ds(base, B // NW)])


def _sc_pair(topo, attr, xl, xr):
    fn = pl.kernel(
        _sc_pair_body,
        out_type=tuple(jax.ShapeDtypeStruct((B, 64), _f32) for _ in range(4)),
        mesh=plsc.VectorSubcoreMesh(core_axis_name="c", subcore_axis_name="s"),
        scratch_types=[
            pltpu.VMEM((B // NW,), _i32),
            pltpu.VMEM((B // NW, 64), _f32),
        ],
        compiler_params=pltpu.CompilerParams(needs_layout_passes=False,
                                             use_tc_tiling_on_sc=False),
    )
    return fn(topo, attr, xl, xr)


_NB = 2000


def _feat_body(x_r, w_r, as_r, ad_r, *rest):
    f_rs = rest[0:8]
    s_r, d_r = rest[8], rest[9]
    f = jnp.dot(x_r[...], w_r[...], preferred_element_type=_f32)
    for g in range(8):
        f_rs[g][...] = f[:, 16 * g:16 * (g + 1)]
    s_r[...] = jnp.dot(f, as_r[...], preferred_element_type=_f32)
    d_r[...] = jnp.dot(f, ad_r[...], preferred_element_type=_f32)


def _tc_feat(x, w, a_s, a_d):
    fin = x.shape[1]
    outs = pl.pallas_call(
        _feat_body,
        grid=(N // _NB,),
        in_specs=[
            pl.BlockSpec((_NB, fin), lambda i: (i, 0)),
            pl.BlockSpec((fin, 128), lambda i: (0, 0)),
            pl.BlockSpec((128, 2), lambda i: (0, 0)),
            pl.BlockSpec((128, 2), lambda i: (0, 0)),
        ],
        out_specs=[pl.BlockSpec((_NB, 16), lambda i: (i, 0))] * 8
        + [pl.BlockSpec((_NB, 2), lambda i: (i, 0))] * 2,
        out_shape=[jax.ShapeDtypeStruct((N, 16), _f32)] * 8
        + [jax.ShapeDtypeStruct((N, 2), _f32)] * 2,
    )(x, w, a_s, a_d)
    return outs[0:8], outs[8], outs[9]


def _epi_body(acc_r, den_r, *rest):
    f_rs = rest[0:8]
    s_r, d_r, b_r, o_r = rest[8], rest[9], rest[10], rest[11]
    a = s_r[...] + d_r[...]
    exs = jnp.exp(jnp.where(a > 0, a, 0.2 * a))
    den = den_r[0] + den_r[1] + exs + 1e-16
    for q in range(4):
        n0 = acc_r[q] + acc_r[8 + q] + f_rs[q][...] * exs[:, 0:1]
        n1 = acc_r[4 + q] + acc_r[12 + q] + f_rs[4 + q][...] * exs[:, 1:2]
        o_r[:, q * 16:(q + 1) * 16] = (
            0.5 * (n0 / den[:, 0:1] + n1 / den[:, 1:2])
            + b_r[q * 16:(q + 1) * 16])


def _tc_epi(acc, den, fcs, asrc, adst, bias):
    nb = 1000
    return pl.pallas_call(
        _epi_body,
        grid=(N // nb,),
        in_specs=[
            pl.BlockSpec((16, nb, 16), lambda i: (0, i, 0)),
            pl.BlockSpec((2, nb, 2), lambda i: (0, i, 0)),
        ]
        + [pl.BlockSpec((nb, 16), lambda i: (i, 0))] * 8
        + [pl.BlockSpec((nb, 2), lambda i: (i, 0))] * 2
        + [pl.BlockSpec((64,), lambda i: (0,))],
        out_specs=pl.BlockSpec((nb, 64), lambda i: (i, 0)),
        out_shape=jax.ShapeDtypeStruct((N, 64), _f32),
    )(acc, den, *fcs, asrc, adst, bias)


def _mlp2_body(x_r, w1_r, b1_r, w2_r, b2_r, w3_r, b3_r, w4_r, b4_r,
               y2_r, y4_r):
    t = jnp.dot(x_r[...], w1_r[...], preferred_element_type=_f32) + b1_r[...]
    y2 = jnp.dot(t, w2_r[...], preferred_element_type=_f32) + b2_r[...]
    y2_r[...] = y2
    u = jnp.dot(y2, w3_r[...], preferred_element_type=_f32) + b3_r[...]
    y4_r[...] = jnp.dot(u, w4_r[...], preferred_element_type=_f32) + b4_r[...]


def _tc_mlp2(x, w1, b1, w2, b2, w3, b3, w4, b4, dmid, dout):
    fin = x.shape[1]
    return pl.pallas_call(
        _mlp2_body,
        grid=(N // _NB,),
        in_specs=[
            pl.BlockSpec((_NB, fin), lambda i: (i, 0)),
            pl.BlockSpec((fin, 100), lambda i: (0, 0)),
            pl.BlockSpec((100,), lambda i: (0,)),
            pl.BlockSpec((100, dmid), lambda i: (0, 0)),
            pl.BlockSpec((dmid,), lambda i: (0,)),
            pl.BlockSpec((dmid, 100), lambda i: (0, 0)),
            pl.BlockSpec((100,), lambda i: (0,)),
            pl.BlockSpec((100, dout), lambda i: (0, 0)),
            pl.BlockSpec((dout,), lambda i: (0,)),
        ],
        out_specs=[
            pl.BlockSpec((_NB, dmid), lambda i: (i, 0)),
            pl.BlockSpec((_NB, dout), lambda i: (i, 0)),
        ],
        out_shape=[
            jax.ShapeDtypeStruct((N, dmid), _f32),
            jax.ShapeDtypeStruct((N, dout), _f32),
        ],
    )(x, w1, b1, w2, b2, w3, b3, w4, b4)


_BB = 2048


def _cls_body(tl_r, al_r, tr_r, ar_r, w1a_r, w1b_r, w1c_r, w1d_r, b1_r,
              g1_r, be1_r, w2_r, b2_r, g2_r, be2_r, w3_r, b3_r, o_r):
    s = 1.0 / jnp.sqrt(1.0 + 1e-5)
    h = (jnp.dot(tl_r[...], w1a_r[...], preferred_element_type=_f32)
         + jnp.dot(al_r[...], w1b_r[...], preferred_element_type=_f32)
         + jnp.dot(tr_r[...], w1c_r[...], preferred_element_type=_f32)
         + jnp.dot(ar_r[...], w1d_r[...], preferred_element_type=_f32)
         + b1_r[...])
    h = jnp.maximum(h, 0.0) * s * g1_r[...] + be1_r[...]
    h = jnp.dot(h, w2_r[...], preferred_element_type=_f32) + b2_r[...]
    h = jnp.maximum(h, 0.0) * s * g2_r[...] + be2_r[...]
    o_r[...] = jax.nn.sigmoid(
        jnp.dot(h, w3_r[...], preferred_element_type=_f32) + b3_r[...])


def _tc_cls(tl, al, tr, ar, p):
    w1 = p['Wd1']
    return pl.pallas_call(
        _cls_body,
        grid=(B // _BB,),
        in_specs=[pl.BlockSpec((_BB, 64), lambda i: (i, 0))] * 4
        + [pl.BlockSpec((64, 512), lambda i: (0, 0))] * 4
        + [pl.BlockSpec((512,), lambda i: (0,))] * 3
        + [pl.BlockSpec((512, 256), lambda i: (0, 0))]
        + [pl.BlockSpec((256,), lambda i: (0,))] * 3
        + [pl.BlockSpec((256, 1), lambda i: (0, 0)),
           pl.BlockSpec((1,), lambda i: (0,))],
        out_specs=pl.BlockSpec((_BB, 1), lambda i: (i, 0)),
        out_shape=jax.ShapeDtypeStruct((B, 1), _f32),
    )(tl, al, tr, ar, w1[0:64], w1[64:128], w1[128:192], w1[192:256],
      p['bd1'], p['g1'], p['be1'], p['Wd2'], p['bd2'], p['g2'], p['be2'],
      p['Wd3'], p['bd3'])


def _alpha_mats(a_src, a_dst):
    a_s = jnp.zeros((128, 2), _f32).at[0:64, 0].set(a_src[0]).at[64:128, 1].set(a_src[1])
    a_d = jnp.zeros((128, 2), _f32).at[0:64, 0].set(a_dst[0]).at[64:128, 1].set(a_dst[1])
    return a_s, a_d


def _gat_layer(x, srcp, dstp, w, a_src, a_dst, bias, z2, z16):
    a_s, a_d = _alpha_mats(a_src, a_dst)
    fcs, asrc, adst = _tc_feat(x, w, a_s, a_d)
    den, acc = _sc_gat(srcp, dstp, asrc.reshape(-1), adst.reshape(-1),
                       fcs, z2, z16)
    return _tc_epi(acc, den.reshape(2, N, 2), fcs, asrc, adst, bias)


def kernel(edge_index, attr_mtx, x_pairs, p):
    pad = jnp.zeros((EP - E,), _i32)
    srcp = jnp.concatenate([edge_index[0], pad])
    dstp = jnp.concatenate([edge_index[1], pad])
    xl = x_pairs[:, 0]
    xr = x_pairs[:, 1]
    z2 = jnp.zeros((2 * N,), _f32)
    z16 = jnp.zeros((N, 16), _f32)

    h1 = _gat_layer(p['X'], srcp, dstp, p['W1'], p['as1'], p['ad1'],
                    p['b1'], z2, z16)
    gcn_out = _gat_layer(h1, srcp, dstp, p['W2'], p['as2'], p['ad2'],
                         p['b2'], z2, z16)

    attr_emb, a2t = _tc_mlp2(attr_mtx, p['Wa1'], p['ba1'], p['Wa2'],
                             p['ba2'], p['Wat1'], p['bat1'], p['Wat2'],
                             p['bat2'], 64, 64)
    topo, t2a = _tc_mlp2(gcn_out, p['Wt1'], p['bt1'], p['Wt2'], p['bt2'],
                         p['Wta1'], p['bta1'], p['Wta2'], p['bta2'], 64, 128)

    tl, al, tr, ar = _sc_pair(topo, attr_emb, xl, xr)
    out = _tc_cls(tl, al, tr, ar, p)
    return (out, gcn_out, t2a, a2t)


# speedup vs baseline: 36.9452x; 1.0001x over previous
import jax
import jax.numpy as jnp
from jax import lax
from jax.experimental import pallas as pl
from jax.experimental.pallas import tpu as pltpu
from jax.experimental.pallas import tpu_sc as plsc

N = 50000
E = 800000
B = 16384
EP = 819200
NW = 32
EW = EP // NW
BK = 800
NBLK = EW // BK

_f32 = jnp.float32
_i32 = jnp.int32


def _sc_gat_body(src_h, dst_h, asrc_h, adst_h, f0_h, f1_h, f2_h, f3_h,
                 f4_h, f5_h, f6_h, f7_h, z2_h, z16_h,
                 den_o, acc_o, ex_o,
                 den_sp, acc_sp, src_i, dst_i, vsrc_i, vdst_i, sg, dg,
                 exblk, rows, src_i2, dst_i2, exblk2, rows2,
                 vsrc_i2, vdst_i2, sg2, dg2,
                 semg0, semg1, semsc0, semsc1,
                 sema0, sema1, semd0, semd1, seme0, seme1):
    c = lax.axis_index("c")
    s = lax.axis_index("s")
    wid = s * 2 + c
    lane = jnp.arange(16, dtype=_i32)
    half = lane >> 1
    par = lane & 1

    src_b = (src_i, src_i2)
    dst_b = (dst_i, dst_i2)
    vsrc_b = (vsrc_i, vsrc_i2)
    vdst_b = (vdst_i, vdst_i2)
    sg_b = (sg, sg2)
    dg_b = (dg, dg2)
    exb_b = (exblk, exblk2)
    sema = (sema0, sema1)
    semd = (semd0, semd1)
    seme = (seme0, seme1)

    @pl.when(s == 0)
    def _():
        pltpu.sync_copy(z2_h, den_sp)
    plsc.subcore_barrier()

    def load_a(blk, bi):
        ebase = wid * EW + blk * BK
        pltpu.sync_copy(src_h.at[pl.ds(ebase, BK)], src_b[bi])
        pltpu.sync_copy(dst_h.at[pl.ds(ebase, BK)], dst_b[bi])

        def mkidx(i, carry2):
            eloc = i * 8 + half
            vsrc_b[bi][pl.ds(i * 16, 16)] = (
                plsc.load_gather(src_b[bi], [eloc]) * 2 + par)
            vdst_b[bi][pl.ds(i * 16, 16)] = (
                plsc.load_gather(dst_b[bi], [eloc]) * 2 + par)
            return carry2
        lax.fori_loop(0, 2 * BK // 16, mkidx, 0)
        pltpu.async_copy(asrc_h.at[vsrc_b[bi]], sg_b[bi], sema[bi])
        pltpu.async_copy(adst_h.at[vdst_b[bi]], dg_b[bi], sema[bi])

    def proc_a(blk, bi):
        nbi = 1 - bi

        @pl.when(blk + 1 < NBLK)
        def _():
            @pl.when(blk >= 1)
            def _():
                pltpu.make_async_copy(
                    exb_b[nbi], den_sp.at[vdst_b[nbi]], semd[nbi]).wait()
                pltpu.make_async_copy(
                    exb_b[nbi], ex_o.at[pl.ds(0, 2 * BK)], seme[nbi]).wait()
            load_a(blk + 1, nbi)

        ebase = wid * EW + blk * BK
        pltpu.make_async_copy(asrc_h.at[vsrc_b[bi]], sg_b[bi],
                              sema[bi]).wait()
        pltpu.make_async_copy(adst_h.at[vdst_b[bi]], dg_b[bi],
                              sema[bi]).wait()

        def cmp16(i, carry2):
            sl = pl.ds(i * 16, 16)
            a = sg_b[bi][sl] + dg_b[bi][sl]
            a = jnp.where(a > 0, a, 0.2 * a)
            ev = jnp.exp(a)
            ev = jnp.where(ebase + i * 8 + half < E, ev, 0.0)
            exb_b[bi][sl] = ev
            return carry2
        lax.fori_loop(0, 2 * BK // 16, cmp16, 0)
        pltpu.async_copy(exb_b[bi], den_sp.at[vdst_b[bi]], semd[bi],
                         add=True)
        pltpu.async_copy(exb_b[bi], ex_o.at[pl.ds(2 * ebase, 2 * BK)],
                         seme[bi])

    load_a(0, 0)

    def pair_a(j, carry):
        proc_a(2 * j, 0)
        proc_a(2 * j + 1, 1)
        return carry
    lax.fori_loop(0, NBLK // 2, pair_a, 0)
    for bi in range(2):
        pltpu.make_async_copy(exb_b[bi], den_sp.at[vdst_b[bi]],
                              semd[bi]).wait()
        pltpu.make_async_copy(exb_b[bi], ex_o.at[pl.ds(0, 2 * BK)],
                              seme[bi]).wait()

    plsc.subcore_barrier()

    @pl.when(s == 0)
    def _():
        pltpu.sync_copy(den_sp, den_o.at[c])

    rows_b = (rows, rows2)
    semg = (semg0, semg1)
    semsc = (semsc0, semsc1)
    semi = (sema0, sema1)

    for chunk in range(8):
        h = chunk // 4
        fc_h = (f0_h, f1_h, f2_h, f3_h, f4_h, f5_h, f6_h, f7_h)[chunk]

        @pl.when(s == 0)
        def _():
            pltpu.sync_copy(z16_h, acc_sp)
        plsc.subcore_barrier()

        def load_blk(blk, bi, fc_h=fc_h):
            ebase = wid * EW + blk * BK
            pltpu.async_copy(src_h.at[pl.ds(ebase, BK)], src_b[bi],
                             semi[bi])
            pltpu.make_async_copy(src_h.at[pl.ds(ebase, BK)], src_b[bi],
                                  semi[bi]).wait()
            pltpu.async_copy(fc_h.at[src_b[bi]], rows_b[bi], semg[bi])
            pltpu.async_copy(dst_h.at[pl.ds(ebase, BK)], dst_b[bi],
                             semi[bi])
            pltpu.async_copy(ex_o.at[pl.ds(2 * ebase, 2 * BK)], exb_b[bi],
                             semi[bi])

        def proc(blk, bi, fc_h=fc_h, h=h):
            nbi = 1 - bi

            @pl.when(blk + 1 < NBLK)
            def _():
                @pl.when(blk >= 1)
                def _():
                    pltpu.make_async_copy(
                        rows_b[nbi], acc_sp.at[dst_b[nbi]],
                        semsc[nbi]).wait()
                load_blk(blk + 1, nbi)

            pltpu.make_async_copy(dst_h.at[pl.ds(0, BK)], dst_b[bi],
                                  semi[bi]).wait()
            pltpu.make_async_copy(ex_o.at[pl.ds(0, 2 * BK)], exb_b[bi],
                                  semi[bi]).wait()
            pltpu.make_async_copy(fc_h.at[src_b[bi]], rows_b[bi],
                                  semg[bi]).wait()
            rr = rows_b[bi]
            ee = exb_b[bi]

            def edge8(j, carry2):
                e0 = j * 8
                for u in range(8):
                    e = e0 + u
                    exv = plsc.load_gather(
                        ee, [jnp.full((16,), 2 * e + h, dtype=_i32)])
                    rr[e, :] = rr[e, :] * exv
                return carry2
            lax.fori_loop(0, BK // 8, edge8, 0)
            pltpu.async_copy(rows_b[bi], acc_sp.at[dst_b[bi]], semsc[bi],
                             add=True)

        load_blk(0, 0)

        def pair(j, carry):
            proc(2 * j, 0)
            proc(2 * j + 1, 1)
            return carry
        lax.fori_loop(0, NBLK // 2, pair, 0)
        pltpu.make_async_copy(rows_b[0], acc_sp.at[dst_b[0]], semsc[0]).wait()
        pltpu.make_async_copy(rows_b[1], acc_sp.at[dst_b[1]], semsc[1]).wait()

        plsc.subcore_barrier()

        @pl.when(s == 0)
        def _(chunk=chunk):
            pltpu.sync_copy(acc_sp, acc_o.at[c * 8 + chunk])


def _sc_gat(srcp, dstp, asrc, adst, fcs, z2, z16):
    fn = pl.kernel(
        _sc_gat_body,
        out_type=(jax.ShapeDtypeStruct((2, 2 * N), _f32),
                  jax.ShapeDtypeStruct((16, N, 16), _f32),
                  jax.ShapeDtypeStruct((2 * EP,), _f32)),
        mesh=plsc.VectorSubcoreMesh(core_axis_name="c", subcore_axis_name="s"),
        scratch_types=[
            pltpu.VMEM_SHARED((2 * N,), _f32),
            pltpu.VMEM_SHARED((N, 16), _f32),
            pltpu.VMEM((BK,), _i32),
            pltpu.VMEM((BK,), _i32),
            pltpu.VMEM((2 * BK,), _i32),
            pltpu.VMEM((2 * BK,), _i32),
            pltpu.VMEM((2 * BK,), _f32),
            pltpu.VMEM((2 * BK,), _f32),
            pltpu.VMEM((2 * BK,), _f32),
            pltpu.VMEM((BK, 16), _f32),
            pltpu.VMEM((BK,), _i32),
            pltpu.VMEM((BK,), _i32),
            pltpu.VMEM((2 * BK,), _f32),
            pltpu.VMEM((BK, 16), _f32),
            pltpu.VMEM((2 * BK,), _i32),
            pltpu.VMEM((2 * BK,), _i32),
            pltpu.VMEM((2 * BK,), _f32),
            pltpu.VMEM((2 * BK,), _f32),
        ] + [pltpu.SemaphoreType.DMA] * 10,
        compiler_params=pltpu.CompilerParams(needs_layout_passes=False,
                                             use_tc_tiling_on_sc=False),
    )
    den, acc, _ex = fn(srcp, dstp, asrc, adst, *fcs, z2, z16)
    return den, acc


def _sc_pair_body(topo_h, attr_h, xl_h, xr_h, tl_o, al_o, tr_o, ar_o,
                  pidx, prow):
    c = lax.axis_index("c")
    s = lax.axis_index("s")
    wid = s * 2 + c
    base = wid * (B // NW)
    pltpu.sync_copy(xl_h.at[pl.ds(base, B // NW)], pidx)
    pltpu.sync_copy(topo_h.at[pidx], prow)
    pltpu.sync_copy(prow, tl_o.at[pl.ds(base, B // NW)])
    pltpu.sync_copy(attr_h.at[pidx], prow)
    pltpu.sync_copy(prow, al_o.at[pl.ds(base, B // NW)])
    pltpu.sync_copy(xr_h.at[pl.ds(base, B // NW)], pidx)
    pltpu.sync_copy(topo_h.at[pidx], prow)
    pltpu.sync_copy(prow, tr_o.at[pl.ds(base, B // NW)])
    pltpu.sync_copy(attr_h.at[pidx], prow)
    pltpu.sync_copy(prow, ar_o.at[pl.ds(base, B // NW)])


def _sc_pair(topo, attr, xl, xr):
    fn = pl.kernel(
        _sc_pair_body,
        out_type=tuple(jax.ShapeDtypeStruct((B, 64), _f32) for _ in range(4)),
        mesh=plsc.VectorSubcoreMesh(core_axis_name="c", subcore_axis_name="s"),
        scratch_types=[
            pltpu.VMEM((B // NW,), _i32),
            pltpu.VMEM((B // NW, 64), _f32),
        ],
        compiler_params=pltpu.CompilerParams(needs_layout_passes=False,
                                             use_tc_tiling_on_sc=False),
    )
    return fn(topo, attr, xl, xr)


_NB = 2000


def _feat_body(x_r, w_r, as_r, ad_r, *rest):
    f_rs = rest[0:8]
    s_r, d_r = rest[8], rest[9]
    f = jnp.dot(x_r[...], w_r[...], preferred_element_type=_f32)
    for g in range(8):
        f_rs[g][...] = f[:, 16 * g:16 * (g + 1)]
    s_r[...] = jnp.dot(f, as_r[...], preferred_element_type=_f32)
    d_r[...] = jnp.dot(f, ad_r[...], preferred_element_type=_f32)


def _tc_feat(x, w, a_s, a_d):
    fin = x.shape[1]
    outs = pl.pallas_call(
        _feat_body,
        grid=(N // _NB,),
        in_specs=[
            pl.BlockSpec((_NB, fin), lambda i: (i, 0)),
            pl.BlockSpec((fin, 128), lambda i: (0, 0)),
            pl.BlockSpec((128, 2), lambda i: (0, 0)),
            pl.BlockSpec((128, 2), lambda i: (0, 0)),
        ],
        out_specs=[pl.BlockSpec((_NB, 16), lambda i: (i, 0))] * 8
        + [pl.BlockSpec((_NB, 2), lambda i: (i, 0))] * 2,
        out_shape=[jax.ShapeDtypeStruct((N, 16), _f32)] * 8
        + [jax.ShapeDtypeStruct((N, 2), _f32)] * 2,
    )(x, w, a_s, a_d)
    return outs[0:8], outs[8], outs[9]


def _epi_body(acc_r, den_r, *rest):
    f_rs = rest[0:8]
    s_r, d_r, b_r, o_r = rest[8], rest[9], rest[10], rest[11]
    a = s_r[...] + d_r[...]
    exs = jnp.exp(jnp.where(a > 0, a, 0.2 * a))
    den = den_r[0] + den_r[1] + exs + 1e-16
    for q in range(4):
        n0 = acc_r[q] + acc_r[8 + q] + f_rs[q][...] * exs[:, 0:1]
        n1 = acc_r[4 + q] + acc_r[12 + q] + f_rs[4 + q][...] * exs[:, 1:2]
        o_r[:, q * 16:(q + 1) * 16] = (
            0.5 * (n0 / den[:, 0:1] + n1 / den[:, 1:2])
            + b_r[q * 16:(q + 1) * 16])


def _tc_epi(acc, den, fcs, asrc, adst, bias):
    nb = 1000
    return pl.pallas_call(
        _epi_body,
        grid=(N // nb,),
        in_specs=[
            pl.BlockSpec((16, nb, 16), lambda i: (0, i, 0)),
            pl.BlockSpec((2, nb, 2), lambda i: (0, i, 0)),
        ]
        + [pl.BlockSpec((nb, 16), lambda i: (i, 0))] * 8
        + [pl.BlockSpec((nb, 2), lambda i: (i, 0))] * 2
        + [pl.BlockSpec((64,), lambda i: (0,))],
        out_specs=pl.BlockSpec((nb, 64), lambda i: (i, 0)),
        out_shape=jax.ShapeDtypeStruct((N, 64), _f32),
    )(acc, den, *fcs, asrc, adst, bias)


def _mlp2_body(x_r, w1_r, b1_r, w2_r, b2_r, w3_r, b3_r, w4_r, b4_r,
               y2_r, y4_r):
    t = jnp.dot(x_r[...], w1_r[...], preferred_element_type=_f32) + b1_r[...]
    y2 = jnp.dot(t, w2_r[...], preferred_element_type=_f32) + b2_r[...]
    y2_r[...] = y2
    u = jnp.dot(y2, w3_r[...], preferred_element_type=_f32) + b3_r[...]
    y4_r[...] = jnp.dot(u, w4_r[...], preferred_element_type=_f32) + b4_r[...]


def _tc_mlp2(x, w1, b1, w2, b2, w3, b3, w4, b4, dmid, dout):
    fin = x.shape[1]
    return pl.pallas_call(
        _mlp2_body,
        grid=(N // _NB,),
        in_specs=[
            pl.BlockSpec((_NB, fin), lambda i: (i, 0)),
            pl.BlockSpec((fin, 100), lambda i: (0, 0)),
            pl.BlockSpec((100,), lambda i: (0,)),
            pl.BlockSpec((100, dmid), lambda i: (0, 0)),
            pl.BlockSpec((dmid,), lambda i: (0,)),
            pl.BlockSpec((dmid, 100), lambda i: (0, 0)),
            pl.BlockSpec((100,), lambda i: (0,)),
            pl.BlockSpec((100, dout), lambda i: (0, 0)),
            pl.BlockSpec((dout,), lambda i: (0,)),
        ],
        out_specs=[
            pl.BlockSpec((_NB, dmid), lambda i: (i, 0)),
            pl.BlockSpec((_NB, dout), lambda i: (i, 0)),
        ],
        out_shape=[
            jax.ShapeDtypeStruct((N, dmid), _f32),
            jax.ShapeDtypeStruct((N, dout), _f32),
        ],
    )(x, w1, b1, w2, b2, w3, b3, w4, b4)


_BB = 2048


def _cls_body(tl_r, al_r, tr_r, ar_r, w1a_r, w1b_r, w1c_r, w1d_r, b1_r,
              g1_r, be1_r, w2_r, b2_r, g2_r, be2_r, w3_r, b3_r, o_r):
    s = 1.0 / jnp.sqrt(1.0 + 1e-5)
    h = (jnp.dot(tl_r[...], w1a_r[...], preferred_element_type=_f32)
         + jnp.dot(al_r[...], w1b_r[...], preferred_element_type=_f32)
         + jnp.dot(tr_r[...], w1c_r[...], preferred_element_type=_f32)
         + jnp.dot(ar_r[...], w1d_r[...], preferred_element_type=_f32)
         + b1_r[...])
    h = jnp.maximum(h, 0.0) * s * g1_r[...] + be1_r[...]
    h = jnp.dot(h, w2_r[...], preferred_element_type=_f32) + b2_r[...]
    h = jnp.maximum(h, 0.0) * s * g2_r[...] + be2_r[...]
    o_r[...] = jax.nn.sigmoid(
        jnp.dot(h, w3_r[...], preferred_element_type=_f32) + b3_r[...])


def _tc_cls(tl, al, tr, ar, p):
    w1 = p['Wd1']
    return pl.pallas_call(
        _cls_body,
        grid=(B // _BB,),
        in_specs=[pl.BlockSpec((_BB, 64), lambda i: (i, 0))] * 4
        + [pl.BlockSpec((64, 512), lambda i: (0, 0))] * 4
        + [pl.BlockSpec((512,), lambda i: (0,))] * 3
        + [pl.BlockSpec((512, 256), lambda i: (0, 0))]
        + [pl.BlockSpec((256,), lambda i: (0,))] * 3
        + [pl.BlockSpec((256, 1), lambda i: (0, 0)),
           pl.BlockSpec((1,), lambda i: (0,))],
        out_specs=pl.BlockSpec((_BB, 1), lambda i: (i, 0)),
        out_shape=jax.ShapeDtypeStruct((B, 1), _f32),
    )(tl, al, tr, ar, w1[0:64], w1[64:128], w1[128:192], w1[192:256],
      p['bd1'], p['g1'], p['be1'], p['Wd2'], p['bd2'], p['g2'], p['be2'],
      p['Wd3'], p['bd3'])


def _alpha_mats(a_src, a_dst):
    a_s = jnp.zeros((128, 2), _f32).at[0:64, 0].set(a_src[0]).at[64:128, 1].set(a_src[1])
    a_d = jnp.zeros((128, 2), _f32).at[0:64, 0].set(a_dst[0]).at[64:128, 1].set(a_dst[1])
    return a_s, a_d


def _gat_layer(x, srcp, dstp, w, a_src, a_dst, bias, z2, z16):
    a_s, a_d = _alpha_mats(a_src, a_dst)
    fcs, asrc, adst = _tc_feat(x, w, a_s, a_d)
    den, acc = _sc_gat(srcp, dstp, asrc.reshape(-1), adst.reshape(-1),
                       fcs, z2, z16)
    return _tc_epi(acc, den.reshape(2, N, 2), fcs, asrc, adst, bias)


def kernel(edge_index, attr_mtx, x_pairs, p):
    pad = jnp.zeros((EP - E,), _i32)
    srcp = jnp.concatenate([edge_index[0], pad])
    dstp = jnp.concatenate([edge_index[1], pad])
    xl = x_pairs[:, 0]
    xr = x_pairs[:, 1]
    z2 = jnp.zeros((2 * N,), _f32)
    z16 = jnp.zeros((N, 16), _f32)

    h1 = _gat_layer(p['X'], srcp, dstp, p['W1'], p['as1'], p['ad1'],
                    p['b1'], z2, z16)
    gcn_out = _gat_layer(h1, srcp, dstp, p['W2'], p['as2'], p['ad2'],
                         p['b2'], z2, z16)

    attr_emb, a2t = _tc_mlp2(attr_mtx, p['Wa1'], p['ba1'], p['Wa2'],
                             p['ba2'], p['Wat1'], p['bat1'], p['Wat2'],
                             p['bat2'], 64, 64)
    topo, t2a = _tc_mlp2(gcn_out, p['Wt1'], p['bt1'], p['Wt2'], p['bt2'],
                         p['Wta1'], p['bta1'], p['Wta2'], p['bta2'], 64, 128)

    tl, al, tr, ar = _sc_pair(topo, attr_emb, xl, xr)
    out = _tc_cls(tl, al, tr, ar, p)
    return (out, gcn_out, t2a, a2t)


# speedup vs baseline: 37.0637x; 1.0032x over previous
import jax
import jax.numpy as jnp
from jax import lax
from jax.experimental import pallas as pl
from jax.experimental.pallas import tpu as pltpu
from jax.experimental.pallas import tpu_sc as plsc

N = 50000
E = 800000
B = 16384
EP = 819200
NW = 32
EW = EP // NW
BK = 1280
NBLK = EW // BK

_f32 = jnp.float32
_i32 = jnp.int32


def _sc_gat_body(src_h, dst_h, asrc_h, adst_h, f0_h, f1_h, f2_h, f3_h,
                 f4_h, f5_h, f6_h, f7_h, z2_h, z16_h,
                 den_o, acc_o, ex_o,
                 den_sp, acc_sp, src_i, dst_i, vsrc_i, vdst_i, sg, dg,
                 exblk, rows, src_i2, dst_i2, exblk2, rows2,
                 vsrc_i2, vdst_i2, sg2, dg2,
                 semg0, semg1, semsc0, semsc1,
                 sema0, sema1, semd0, semd1, seme0, seme1):
    c = lax.axis_index("c")
    s = lax.axis_index("s")
    wid = s * 2 + c
    lane = jnp.arange(16, dtype=_i32)
    half = lane >> 1
    par = lane & 1

    src_b = (src_i, src_i2)
    dst_b = (dst_i, dst_i2)
    vsrc_b = (vsrc_i, vsrc_i2)
    vdst_b = (vdst_i, vdst_i2)
    sg_b = (sg, sg2)
    dg_b = (dg, dg2)
    exb_b = (exblk, exblk2)
    sema = (sema0, sema1)
    semd = (semd0, semd1)
    seme = (seme0, seme1)

    @pl.when(s == 0)
    def _():
        pltpu.sync_copy(z2_h, den_sp)
    plsc.subcore_barrier()

    def load_a(blk, bi):
        ebase = wid * EW + blk * BK
        pltpu.sync_copy(src_h.at[pl.ds(ebase, BK)], src_b[bi])
        pltpu.sync_copy(dst_h.at[pl.ds(ebase, BK)], dst_b[bi])

        def mkidx(i, carry2):
            eloc = i * 8 + half
            vsrc_b[bi][pl.ds(i * 16, 16)] = (
                plsc.load_gather(src_b[bi], [eloc]) * 2 + par)
            vdst_b[bi][pl.ds(i * 16, 16)] = (
                plsc.load_gather(dst_b[bi], [eloc]) * 2 + par)
            return carry2
        lax.fori_loop(0, 2 * BK // 16, mkidx, 0)
        pltpu.async_copy(asrc_h.at[vsrc_b[bi]], sg_b[bi], sema[bi])
        pltpu.async_copy(adst_h.at[vdst_b[bi]], dg_b[bi], sema[bi])

    def proc_a(blk, bi):
        nbi = 1 - bi

        @pl.when(blk + 1 < NBLK)
        def _():
            @pl.when(blk >= 1)
            def _():
                pltpu.make_async_copy(
                    exb_b[nbi], den_sp.at[vdst_b[nbi]], semd[nbi]).wait()
                pltpu.make_async_copy(
                    exb_b[nbi], ex_o.at[pl.ds(0, 2 * BK)], seme[nbi]).wait()
            load_a(blk + 1, nbi)

        ebase = wid * EW + blk * BK
        pltpu.make_async_copy(asrc_h.at[vsrc_b[bi]], sg_b[bi],
                              sema[bi]).wait()
        pltpu.make_async_copy(adst_h.at[vdst_b[bi]], dg_b[bi],
                              sema[bi]).wait()

        def cmp16(i, carry2):
            sl = pl.ds(i * 16, 16)
            a = sg_b[bi][sl] + dg_b[bi][sl]
            a = jnp.where(a > 0, a, 0.2 * a)
            ev = jnp.exp(a)
            ev = jnp.where(ebase + i * 8 + half < E, ev, 0.0)
            exb_b[bi][sl] = ev
            return carry2
        lax.fori_loop(0, 2 * BK // 16, cmp16, 0)
        pltpu.async_copy(exb_b[bi], den_sp.at[vdst_b[bi]], semd[bi],
                         add=True)
        pltpu.async_copy(exb_b[bi], ex_o.at[pl.ds(2 * ebase, 2 * BK)],
                         seme[bi])

    load_a(0, 0)

    def pair_a(j, carry):
        proc_a(2 * j, 0)
        proc_a(2 * j + 1, 1)
        return carry
    lax.fori_loop(0, NBLK // 2, pair_a, 0)
    for bi in range(2):
        pltpu.make_async_copy(exb_b[bi], den_sp.at[vdst_b[bi]],
                              semd[bi]).wait()
        pltpu.make_async_copy(exb_b[bi], ex_o.at[pl.ds(0, 2 * BK)],
                              seme[bi]).wait()

    plsc.subcore_barrier()

    @pl.when(s == 0)
    def _():
        pltpu.sync_copy(den_sp, den_o.at[c])

    rows_b = (rows, rows2)
    semg = (semg0, semg1)
    semsc = (semsc0, semsc1)
    semi = (sema0, sema1)

    for chunk in range(8):
        h = chunk // 4
        fc_h = (f0_h, f1_h, f2_h, f3_h, f4_h, f5_h, f6_h, f7_h)[chunk]

        @pl.when(s == 0)
        def _():
            pltpu.sync_copy(z16_h, acc_sp)
        plsc.subcore_barrier()

        def load_blk(blk, bi, fc_h=fc_h):
            ebase = wid * EW + blk * BK
            pltpu.async_copy(src_h.at[pl.ds(ebase, BK)], src_b[bi],
                             semi[bi])
            pltpu.make_async_copy(src_h.at[pl.ds(ebase, BK)], src_b[bi],
                                  semi[bi]).wait()
            pltpu.async_copy(fc_h.at[src_b[bi]], rows_b[bi], semg[bi])
            pltpu.async_copy(dst_h.at[pl.ds(ebase, BK)], dst_b[bi],
                             semi[bi])
            pltpu.async_copy(ex_o.at[pl.ds(2 * ebase, 2 * BK)], exb_b[bi],
                             semi[bi])

        def proc(blk, bi, fc_h=fc_h, h=h):
            nbi = 1 - bi

            @pl.when(blk + 1 < NBLK)
            def _():
                @pl.when(blk >= 1)
                def _():
                    pltpu.make_async_copy(
                        rows_b[nbi], acc_sp.at[dst_b[nbi]],
                        semsc[nbi]).wait()
                load_blk(blk + 1, nbi)

            pltpu.make_async_copy(dst_h.at[pl.ds(0, BK)], dst_b[bi],
                                  semi[bi]).wait()
            pltpu.make_async_copy(ex_o.at[pl.ds(0, 2 * BK)], exb_b[bi],
                                  semi[bi]).wait()
            pltpu.make_async_copy(fc_h.at[src_b[bi]], rows_b[bi],
                                  semg[bi]).wait()
            rr = rows_b[bi]
            ee = exb_b[bi]

            def edge8(j, carry2):
                e0 = j * 8
                for u in range(8):
                    e = e0 + u
                    exv = plsc.load_gather(
                        ee, [jnp.full((16,), 2 * e + h, dtype=_i32)])
                    rr[e, :] = rr[e, :] * exv
                return carry2
            lax.fori_loop(0, BK // 8, edge8, 0)
            pltpu.async_copy(rows_b[bi], acc_sp.at[dst_b[bi]], semsc[bi],
                             add=True)

        load_blk(0, 0)

        def pair(j, carry):
            proc(2 * j, 0)
            proc(2 * j + 1, 1)
            return carry
        lax.fori_loop(0, NBLK // 2, pair, 0)
        pltpu.make_async_copy(rows_b[0], acc_sp.at[dst_b[0]], semsc[0]).wait()
        pltpu.make_async_copy(rows_b[1], acc_sp.at[dst_b[1]], semsc[1]).wait()

        plsc.subcore_barrier()

        @pl.when(s == 0)
        def _(chunk=chunk):
            pltpu.sync_copy(acc_sp, acc_o.at[c * 8 + chunk])


def _sc_gat(srcp, dstp, asrc, adst, fcs, z2, z16):
    fn = pl.kernel(
        _sc_gat_body,
        out_type=(jax.ShapeDtypeStruct((2, 2 * N), _f32),
                  jax.ShapeDtypeStruct((16, N, 16), _f32),
                  jax.ShapeDtypeStruct((2 * EP,), _f32)),
        mesh=plsc.VectorSubcoreMesh(core_axis_name="c", subcore_axis_name="s"),
        scratch_types=[
            pltpu.VMEM_SHARED((2 * N,), _f32),
            pltpu.VMEM_SHARED((N, 16), _f32),
            pltpu.VMEM((BK,), _i32),
            pltpu.VMEM((BK,), _i32),
            pltpu.VMEM((2 * BK,), _i32),
            pltpu.VMEM((2 * BK,), _i32),
            pltpu.VMEM((2 * BK,), _f32),
            pltpu.VMEM((2 * BK,), _f32),
            pltpu.VMEM((2 * BK,), _f32),
            pltpu.VMEM((BK, 16), _f32),
            pltpu.VMEM((BK,), _i32),
            pltpu.VMEM((BK,), _i32),
            pltpu.VMEM((2 * BK,), _f32),
            pltpu.VMEM((BK, 16), _f32),
            pltpu.VMEM((2 * BK,), _i32),
            pltpu.VMEM((2 * BK,), _i32),
            pltpu.VMEM((2 * BK,), _f32),
            pltpu.VMEM((2 * BK,), _f32),
        ] + [pltpu.SemaphoreType.DMA] * 10,
        compiler_params=pltpu.CompilerParams(needs_layout_passes=False,
                                             use_tc_tiling_on_sc=False),
    )
    den, acc, _ex = fn(srcp, dstp, asrc, adst, *fcs, z2, z16)
    return den, acc


def _sc_pair_body(topo_h, attr_h, xl_h, xr_h, tl_o, al_o, tr_o, ar_o,
                  pidx, prow):
    c = lax.axis_index("c")
    s = lax.axis_index("s")
    wid = s * 2 + c
    base = wid * (B // NW)
    pltpu.sync_copy(xl_h.at[pl.ds(base, B // NW)], pidx)
    pltpu.sync_copy(topo_h.at[pidx], prow)
    pltpu.sync_copy(prow, tl_o.at[pl.ds(base, B // NW)])
    pltpu.sync_copy(attr_h.at[pidx], prow)
    pltpu.sync_copy(prow, al_o.at[pl.ds(base, B // NW)])
    pltpu.sync_copy(xr_h.at[pl.ds(base, B // NW)], pidx)
    pltpu.sync_copy(topo_h.at[pidx], prow)
    pltpu.sync_copy(prow, tr_o.at[pl.ds(base, B // NW)])
    pltpu.sync_copy(attr_h.at[pidx], prow)
    pltpu.sync_copy(prow, ar_o.at[pl.ds(base, B // NW)])


def _sc_pair(topo, attr, xl, xr):
    fn = pl.kernel(
        _sc_pair_body,
        out_type=tuple(jax.ShapeDtypeStruct((B, 64), _f32) for _ in range(4)),
        mesh=plsc.VectorSubcoreMesh(core_axis_name="c", subcore_axis_name="s"),
        scratch_types=[
            pltpu.VMEM((B // NW,), _i32),
            pltpu.VMEM((B // NW, 64), _f32),
        ],
        compiler_params=pltpu.CompilerParams(needs_layout_passes=False,
                                             use_tc_tiling_on_sc=False),
    )
    return fn(topo, attr, xl, xr)


_NB = 2000


def _feat_body(x_r, w_r, as_r, ad_r, *rest):
    f_rs = rest[0:8]
    s_r, d_r = rest[8], rest[9]
    f = jnp.dot(x_r[...], w_r[...], preferred_element_type=_f32)
    for g in range(8):
        f_rs[g][...] = f[:, 16 * g:16 * (g + 1)]
    s_r[...] = jnp.dot(f, as_r[...], preferred_element_type=_f32)
    d_r[...] = jnp.dot(f, ad_r[...], preferred_element_type=_f32)


def _tc_feat(x, w, a_s, a_d):
    fin = x.shape[1]
    outs = pl.pallas_call(
        _feat_body,
        grid=(N // _NB,),
        in_specs=[
            pl.BlockSpec((_NB, fin), lambda i: (i, 0)),
            pl.BlockSpec((fin, 128), lambda i: (0, 0)),
            pl.BlockSpec((128, 2), lambda i: (0, 0)),
            pl.BlockSpec((128, 2), lambda i: (0, 0)),
        ],
        out_specs=[pl.BlockSpec((_NB, 16), lambda i: (i, 0))] * 8
        + [pl.BlockSpec((_NB, 2), lambda i: (i, 0))] * 2,
        out_shape=[jax.ShapeDtypeStruct((N, 16), _f32)] * 8
        + [jax.ShapeDtypeStruct((N, 2), _f32)] * 2,
    )(x, w, a_s, a_d)
    return outs[0:8], outs[8], outs[9]


def _epi_body(acc_r, den_r, *rest):
    f_rs = rest[0:8]
    s_r, d_r, b_r, o_r = rest[8], rest[9], rest[10], rest[11]
    a = s_r[...] + d_r[...]
    exs = jnp.exp(jnp.where(a > 0, a, 0.2 * a))
    den = den_r[0] + den_r[1] + exs + 1e-16
    for q in range(4):
        n0 = acc_r[q] + acc_r[8 + q] + f_rs[q][...] * exs[:, 0:1]
        n1 = acc_r[4 + q] + acc_r[12 + q] + f_rs[4 + q][...] * exs[:, 1:2]
        o_r[:, q * 16:(q + 1) * 16] = (
            0.5 * (n0 / den[:, 0:1] + n1 / den[:, 1:2])
            + b_r[q * 16:(q + 1) * 16])


def _tc_epi(acc, den, fcs, asrc, adst, bias):
    nb = 1000
    return pl.pallas_call(
        _epi_body,
        grid=(N // nb,),
        in_specs=[
            pl.BlockSpec((16, nb, 16), lambda i: (0, i, 0)),
            pl.BlockSpec((2, nb, 2), lambda i: (0, i, 0)),
        ]
        + [pl.BlockSpec((nb, 16), lambda i: (i, 0))] * 8
        + [pl.BlockSpec((nb, 2), lambda i: (i, 0))] * 2
        + [pl.BlockSpec((64,), lambda i: (0,))],
        out_specs=pl.BlockSpec((nb, 64), lambda i: (i, 0)),
        out_shape=jax.ShapeDtypeStruct((N, 64), _f32),
    )(acc, den, *fcs, asrc, adst, bias)


def _mlp2_body(x_r, w1_r, b1_r, w2_r, b2_r, w3_r, b3_r, w4_r, b4_r,
               y2_r, y4_r):
    t = jnp.dot(x_r[...], w1_r[...], preferred_element_type=_f32) + b1_r[...]
    y2 = jnp.dot(t, w2_r[...], preferred_element_type=_f32) + b2_r[...]
    y2_r[...] = y2
    u = jnp.dot(y2, w3_r[...], preferred_element_type=_f32) + b3_r[...]
    y4_r[...] = jnp.dot(u, w4_r[...], preferred_element_type=_f32) + b4_r[...]


def _tc_mlp2(x, w1, b1, w2, b2, w3, b3, w4, b4, dmid, dout):
    fin = x.shape[1]
    return pl.pallas_call(
        _mlp2_body,
        grid=(N // _NB,),
        in_specs=[
            pl.BlockSpec((_NB, fin), lambda i: (i, 0)),
            pl.BlockSpec((fin, 100), lambda i: (0, 0)),
            pl.BlockSpec((100,), lambda i: (0,)),
            pl.BlockSpec((100, dmid), lambda i: (0, 0)),
            pl.BlockSpec((dmid,), lambda i: (0,)),
            pl.BlockSpec((dmid, 100), lambda i: (0, 0)),
            pl.BlockSpec((100,), lambda i: (0,)),
            pl.BlockSpec((100, dout), lambda i: (0, 0)),
            pl.BlockSpec((dout,), lambda i: (0,)),
        ],
        out_specs=[
            pl.BlockSpec((_NB, dmid), lambda i: (i, 0)),
            pl.BlockSpec((_NB, dout), lambda i: (i, 0)),
        ],
        out_shape=[
            jax.ShapeDtypeStruct((N, dmid), _f32),
            jax.ShapeDtypeStruct((N, dout), _f32),
        ],
    )(x, w1, b1, w2, b2, w3, b3, w4, b4)


_BB = 2048


def _cls_body(tl_r, al_r, tr_r, ar_r, w1a_r, w1b_r, w1c_r, w1d_r, b1_r,
              g1_r, be1_r, w2_r, b2_r, g2_r, be2_r, w3_r, b3_r, o_r):
    s = 1.0 / jnp.sqrt(1.0 + 1e-5)
    h = (jnp.dot(tl_r[...], w1a_r[...], preferred_element_type=_f32)
         + jnp.dot(al_r[...], w1b_r[...], preferred_element_type=_f32)
         + jnp.dot(tr_r[...], w1c_r[...], preferred_element_type=_f32)
         + jnp.dot(ar_r[...], w1d_r[...], preferred_element_type=_f32)
         + b1_r[...])
    h = jnp.maximum(h, 0.0) * s * g1_r[...] + be1_r[...]
    h = jnp.dot(h, w2_r[...], preferred_element_type=_f32) + b2_r[...]
    h = jnp.maximum(h, 0.0) * s * g2_r[...] + be2_r[...]
    o_r[...] = jax.nn.sigmoid(
        jnp.dot(h, w3_r[...], preferred_element_type=_f32) + b3_r[...])


def _tc_cls(tl, al, tr, ar, p):
    w1 = p['Wd1']
    return pl.pallas_call(
        _cls_body,
        grid=(B // _BB,),
        in_specs=[pl.BlockSpec((_BB, 64), lambda i: (i, 0))] * 4
        + [pl.BlockSpec((64, 512), lambda i: (0, 0))] * 4
        + [pl.BlockSpec((512,), lambda i: (0,))] * 3
        + [pl.BlockSpec((512, 256), lambda i: (0, 0))]
        + [pl.BlockSpec((256,), lambda i: (0,))] * 3
        + [pl.BlockSpec((256, 1), lambda i: (0, 0)),
           pl.BlockSpec((1,), lambda i: (0,))],
        out_specs=pl.BlockSpec((_BB, 1), lambda i: (i, 0)),
        out_shape=jax.ShapeDtypeStruct((B, 1), _f32),
    )(tl, al, tr, ar, w1[0:64], w1[64:128], w1[128:192], w1[192:256],
      p['bd1'], p['g1'], p['be1'], p['Wd2'], p['bd2'], p['g2'], p['be2'],
      p['Wd3'], p['bd3'])


def _alpha_mats(a_src, a_dst):
    a_s = jnp.zeros((128, 2), _f32).at[0:64, 0].set(a_src[0]).at[64:128, 1].set(a_src[1])
    a_d = jnp.zeros((128, 2), _f32).at[0:64, 0].set(a_dst[0]).at[64:128, 1].set(a_dst[1])
    return a_s, a_d


def _gat_layer(x, srcp, dstp, w, a_src, a_dst, bias, z2, z16):
    a_s, a_d = _alpha_mats(a_src, a_dst)
    fcs, asrc, adst = _tc_feat(x, w, a_s, a_d)
    den, acc = _sc_gat(srcp, dstp, asrc.reshape(-1), adst.reshape(-1),
                       fcs, z2, z16)
    return _tc_epi(acc, den.reshape(2, N, 2), fcs, asrc, adst, bias)


def kernel(edge_index, attr_mtx, x_pairs, p):
    pad = jnp.zeros((EP - E,), _i32)
    srcp = jnp.concatenate([edge_index[0], pad])
    dstp = jnp.concatenate([edge_index[1], pad])
    xl = x_pairs[:, 0]
    xr = x_pairs[:, 1]
    z2 = jnp.zeros((2 * N,), _f32)
    z16 = jnp.zeros((N, 16), _f32)

    h1 = _gat_layer(p['X'], srcp, dstp, p['W1'], p['as1'], p['ad1'],
                    p['b1'], z2, z16)
    gcn_out = _gat_layer(h1, srcp, dstp, p['W2'], p['as2'], p['ad2'],
                         p['b2'], z2, z16)

    attr_emb, a2t = _tc_mlp2(attr_mtx, p['Wa1'], p['ba1'], p['Wa2'],
                             p['ba2'], p['Wat1'], p['bat1'], p['Wat2'],
                             p['bat2'], 64, 64)
    topo, t2a = _tc_mlp2(gcn_out, p['Wt1'], p['bt1'], p['Wt2'], p['bt2'],
                         p['Wta1'], p['bta1'], p['Wta2'], p['bta2'], 64, 128)

    tl, al, tr, ar = _sc_pair(topo, attr_emb, xl, xr)
    out = _tc_cls(tl, al, tr, ar, p)
    return (out, gcn_out, t2a, a2t)
